# baseline (speedup 1.0000x reference)
import functools

import jax
import jax.numpy as jnp
from jax import lax
from jax.experimental import pallas as pl
from jax.experimental.pallas import tpu as pltpu
from jax.experimental.pallas import tpu_sc as plsc

N_NODES = 10000
N_EDGES = 320000
H = 128
NUM_RBF = 20

NC, NS = 2, 16
NW = NC * NS
EPW = N_EDGES // NW
EPT = N_EDGES // NS
GC = 80
SC_CHUNK = 80

def _mesh():
    return plsc.VectorSubcoreMesh(
        core_axis_name="c", subcore_axis_name="s", num_cores=NC, num_subcores=NS)


_NCH = EPW // GC


@functools.cache
def _gather_stage():
    @functools.partial(
        pl.kernel,
        out_type=[
            jax.ShapeDtypeStruct((N_EDGES, H), jnp.float32),
            jax.ShapeDtypeStruct((N_EDGES, H), jnp.float32),
        ],
        mesh=_mesh(),
        compiler_params=pltpu.CompilerParams(needs_layout_passes=False),
        scratch_types=[
            [pltpu.VMEM((GC,), jnp.int32)] * 2,
            [pltpu.VMEM((GC, H), jnp.float32)] * 2,
            [pltpu.VMEM((GC, 3 * H), jnp.float32)] * 2,
            [pltpu.VMEM((3 * GC,), jnp.float32)] * 2,
            pltpu.VMEM((GC, H), jnp.float32),
            [pltpu.SemaphoreType.DMA] * 2,
            [pltpu.SemaphoreType.DMA] * 2,
            [pltpu.SemaphoreType.DMA] * 2,
        ],
    )
    def gather_k(j_hbm, s_hbm, v_hbm, vecf_hbm, sj_out, inner_out, idx_v,
                 srow_v, vrow_v, vecc_v, ibuf_v, sem_s, sem_v, sem_c):
        wid = lax.axis_index("s") * NC + lax.axis_index("c")
        base = wid * EPW

        def issue(k, b):
            e0 = base + k * GC
            pltpu.sync_copy(j_hbm.at[pl.ds(e0, GC)], idx_v[b])
            pltpu.async_copy(s_hbm.at[idx_v[b]], srow_v[b], sem_s[b])
            pltpu.async_copy(v_hbm.at[idx_v[b]], vrow_v[b], sem_v[b])
            pltpu.async_copy(vecf_hbm.at[pl.ds(3 * e0, 3 * GC)], vecc_v[b],
                             sem_c[b])

        def consume(k, b):
            e0 = base + k * GC
            pltpu.make_async_copy(s_hbm.at[idx_v[b]], srow_v[b],
                                  sem_s[b]).wait()
            pltpu.sync_copy(srow_v[b], sj_out.at[pl.ds(e0, GC), :])
            pltpu.make_async_copy(v_hbm.at[idx_v[b]], vrow_v[b],
                                  sem_v[b]).wait()
            pltpu.make_async_copy(vecf_hbm.at[pl.ds(3 * e0, 3 * GC)],
                                  vecc_v[b], sem_c[b]).wait()

            def edge_body(e, carry):
                cs = [
                    plsc.load_gather(
                        vecc_v[b],
                        [jnp.full((16,), 3 * e + d, jnp.int32)])
                    for d in range(3)
                ]
                for kk in range(H // 16):
                    acc = (vrow_v[b][e, pl.ds(kk * 16, 16)] * cs[0]
                           + vrow_v[b][e, pl.ds(H + kk * 16, 16)] * cs[1]
                           + vrow_v[b][e, pl.ds(2 * H + kk * 16, 16)] * cs[2])
                    ibuf_v[e, pl.ds(kk * 16, 16)] = acc
                return carry

            lax.fori_loop(0, GC, edge_body, 0)
            pltpu.sync_copy(ibuf_v, inner_out.at[pl.ds(e0, GC), :])

        issue(0, 0)

        def body(m, carry):
            k0 = 2 * m

            @pl.when(k0 + 1 < _NCH)
            def _():
                issue(k0 + 1, 1)

            consume(k0, 0)

            @pl.when(k0 + 2 < _NCH)
            def _():
                issue(k0 + 2, 0)

            @pl.when(k0 + 1 < _NCH)
            def _():
                consume(k0 + 1, 1)

            return carry

        lax.fori_loop(0, (_NCH + 1) // 2, body, 0)

    return gather_k


_TCB = 640


def _tc_body(sj_ref, in_ref, rbf_ref, cut_ref, vec_ref, w1_ref, b1_ref,
             w2_ref, b2_ref, wr_ref, br_ref, z_ref):
    sj = sj_ref[...]
    h = jnp.dot(sj, w1_ref[...], preferred_element_type=jnp.float32) + b1_ref[...]
    h = h * (1.0 / (1.0 + jnp.exp(-h)))
    h = jnp.dot(h, w2_ref[...], preferred_element_type=jnp.float32) + b2_ref[...]
    wt = jnp.dot(rbf_ref[...], wr_ref[...], preferred_element_type=jnp.float32)
    wt = (wt + br_ref[...]) * cut_ref[...]
    x = h * wt
    x_ss = x[:, :H]
    x_sv = x[:, H:2 * H]
    x_vv = x[:, 2 * H:]
    vec = vec_ref[...]
    u = x_sv + in_ref[...] * x_vv
    z_ref[0] = x_ss
    z_ref[1] = u * vec[:, 0:1]
    z_ref[2] = u * vec[:, 1:2]
    z_ref[3] = u * vec[:, 2:3]


def _tc_stage(sj, inner, rbf, cut, vec, w1, b1, w2, b2, wr, br):
    grid = (N_EDGES // _TCB,)
    return pl.pallas_call(
        _tc_body,
        grid=grid,
        in_specs=[
            pl.BlockSpec((_TCB, H), lambda e: (e, 0)),
            pl.BlockSpec((_TCB, H), lambda e: (e, 0)),
            pl.BlockSpec((_TCB, NUM_RBF), lambda e: (e, 0)),
            pl.BlockSpec((_TCB, 1), lambda e: (e, 0)),
            pl.BlockSpec((_TCB, 3), lambda e: (e, 0)),
            pl.BlockSpec((H, H), lambda e: (0, 0)),
            pl.BlockSpec((1, H), lambda e: (0, 0)),
            pl.BlockSpec((H, 3 * H), lambda e: (0, 0)),
            pl.BlockSpec((1, 3 * H), lambda e: (0, 0)),
            pl.BlockSpec((NUM_RBF, 3 * H), lambda e: (0, 0)),
            pl.BlockSpec((1, 3 * H), lambda e: (0, 0)),
        ],
        out_specs=pl.BlockSpec((4, _TCB, H), lambda e: (0, e, 0)),
        out_shape=jax.ShapeDtypeStruct((4, N_EDGES, H), jnp.float32),
    )(sj, inner, rbf, cut, vec, w1, b1, w2, b2, wr, br)


@functools.cache
def _scatter_stage():
    @functools.partial(
        pl.kernel,
        out_type=jax.ShapeDtypeStruct((4, N_NODES, H), jnp.float32),
        mesh=_mesh(),
        scratch_types=[
            pltpu.VMEM((SC_CHUNK,), jnp.int32),
            pltpu.VMEM((SC_CHUNK, H), jnp.float32),
            pltpu.VMEM_SHARED((N_NODES, H), jnp.float32),
        ],
    )
    def scatter_k(i_hbm, z_hbm, zero_hbm, out4, idx_v, row_v, table):
        core = lax.axis_index("c")
        sub = lax.axis_index("s")
        for q in range(2):
            p = 2 * core + q

            @pl.when(sub == 0)
            def _zero():
                pltpu.sync_copy(zero_hbm, table)

            plsc.subcore_barrier()

            def body(k, carry):
                e0 = sub * EPT + k * SC_CHUNK
                pltpu.sync_copy(i_hbm.at[pl.ds(e0, SC_CHUNK)], idx_v)
                pltpu.sync_copy(z_hbm.at[p, pl.ds(e0, SC_CHUNK), :], row_v)
                pltpu.sync_copy(row_v, table.at[idx_v], add=True)
                return carry

            lax.fori_loop(0, EPT // SC_CHUNK, body, 0)
            plsc.subcore_barrier()

            @pl.when(sub == 0)
            def _flush():
                pltpu.sync_copy(table, out4.at[p])

            plsc.subcore_barrier()

    return scatter_k


def kernel(s, v, edge_index, edge_rbf, edge_cutoff, edge_vec, W1, b1, W2, b2,
           Wr, br):
    i = edge_index[0].astype(jnp.int32)
    j = edge_index[1].astype(jnp.int32)
    n = s.shape[0]
    v2d = v.reshape(n, 3 * H)
    vecf = edge_vec.reshape(-1)

    sj, inner = _gather_stage()(j, s, v2d, vecf)
    z = _tc_stage(sj, inner, edge_rbf, edge_cutoff[:, None], edge_vec,
                  W1, b1[None, :], W2, b2[None, :], Wr, br[None, :])
    zero = jnp.zeros((n, H), jnp.float32)
    out4 = _scatter_stage()(i, z, zero)
    ds = out4[0]
    dv = jnp.transpose(out4[1:4], (1, 0, 2))
    return ds, dv


# speedup vs baseline: 1.1248x; 1.1248x over previous
import functools

import jax
import jax.numpy as jnp
from jax import lax
from jax.experimental import pallas as pl
from jax.experimental.pallas import tpu as pltpu
from jax.experimental.pallas import tpu_sc as plsc

N_NODES = 10000
N_EDGES = 320000
H = 128
NUM_RBF = 20

NC, NS = 2, 16
NW = NC * NS
EPW = N_EDGES // NW
EPT = N_EDGES // NS
GC = 80
SC_CHUNK = 80

def _mesh():
    return plsc.VectorSubcoreMesh(
        core_axis_name="c", subcore_axis_name="s", num_cores=NC, num_subcores=NS)


_NCH = EPW // GC


@functools.cache
def _gather_stage():
    @functools.partial(
        pl.kernel,
        out_type=[
            jax.ShapeDtypeStruct((N_EDGES, H), jnp.float32),
            jax.ShapeDtypeStruct((N_EDGES, H), jnp.float32),
        ],
        mesh=_mesh(),
        compiler_params=pltpu.CompilerParams(needs_layout_passes=False),
        scratch_types=[
            [pltpu.VMEM((GC,), jnp.int32)] * 2,
            [pltpu.VMEM((GC, H), jnp.float32)] * 2,
            [pltpu.VMEM((GC, 3 * H), jnp.float32)] * 2,
            [pltpu.VMEM((3 * GC,), jnp.float32)] * 2,
            pltpu.VMEM((GC, H), jnp.float32),
            [pltpu.SemaphoreType.DMA] * 2,
            [pltpu.SemaphoreType.DMA] * 2,
            [pltpu.SemaphoreType.DMA] * 2,
        ],
    )
    def gather_k(j_hbm, s_hbm, v_hbm, vecf_hbm, sj_out, inner_out, idx_v,
                 srow_v, vrow_v, vecc_v, ibuf_v, sem_s, sem_v, sem_c):
        wid = lax.axis_index("s") * NC + lax.axis_index("c")
        base = wid * EPW

        def issue(k, b):
            e0 = base + k * GC
            pltpu.sync_copy(j_hbm.at[pl.ds(e0, GC)], idx_v[b])
            pltpu.async_copy(s_hbm.at[idx_v[b]], srow_v[b], sem_s[b])
            pltpu.async_copy(v_hbm.at[idx_v[b]], vrow_v[b], sem_v[b])
            pltpu.async_copy(vecf_hbm.at[pl.ds(3 * e0, 3 * GC)], vecc_v[b],
                             sem_c[b])

        def consume(k, b):
            e0 = base + k * GC
            pltpu.make_async_copy(s_hbm.at[idx_v[b]], srow_v[b],
                                  sem_s[b]).wait()
            pltpu.sync_copy(srow_v[b], sj_out.at[pl.ds(e0, GC), :])
            pltpu.make_async_copy(v_hbm.at[idx_v[b]], vrow_v[b],
                                  sem_v[b]).wait()
            pltpu.make_async_copy(vecf_hbm.at[pl.ds(3 * e0, 3 * GC)],
                                  vecc_v[b], sem_c[b]).wait()

            def edge_body(e, carry):
                cs = [
                    plsc.load_gather(
                        vecc_v[b],
                        [jnp.full((16,), 3 * e + d, jnp.int32)])
                    for d in range(3)
                ]
                for kk in range(H // 16):
                    acc = (vrow_v[b][e, pl.ds(kk * 16, 16)] * cs[0]
                           + vrow_v[b][e, pl.ds(H + kk * 16, 16)] * cs[1]
                           + vrow_v[b][e, pl.ds(2 * H + kk * 16, 16)] * cs[2])
                    ibuf_v[e, pl.ds(kk * 16, 16)] = acc
                return carry

            lax.fori_loop(0, GC, edge_body, 0)
            pltpu.sync_copy(ibuf_v, inner_out.at[pl.ds(e0, GC), :])

        issue(0, 0)

        def body(m, carry):
            k0 = 2 * m

            @pl.when(k0 + 1 < _NCH)
            def _():
                issue(k0 + 1, 1)

            consume(k0, 0)

            @pl.when(k0 + 2 < _NCH)
            def _():
                issue(k0 + 2, 0)

            @pl.when(k0 + 1 < _NCH)
            def _():
                consume(k0 + 1, 1)

            return carry

        lax.fori_loop(0, (_NCH + 1) // 2, body, 0)

    return gather_k


_TCB = 640


def _tc_body(sj_ref, in_ref, rbf_ref, cut_ref, vec_ref, w1_ref, b1_ref,
             w2_ref, b2_ref, wr_ref, br_ref, z_ref):
    sj = sj_ref[...]
    h = jnp.dot(sj, w1_ref[...], preferred_element_type=jnp.float32) + b1_ref[...]
    h = h * (1.0 / (1.0 + jnp.exp(-h)))
    h = jnp.dot(h, w2_ref[...], preferred_element_type=jnp.float32) + b2_ref[...]
    wt = jnp.dot(rbf_ref[...], wr_ref[...], preferred_element_type=jnp.float32)
    wt = (wt + br_ref[...]) * cut_ref[...]
    x = h * wt
    x_ss = x[:, :H]
    x_sv = x[:, H:2 * H]
    x_vv = x[:, 2 * H:]
    vec = vec_ref[...]
    u = x_sv + in_ref[...] * x_vv
    z_ref[0] = x_ss
    z_ref[1] = u * vec[:, 0:1]
    z_ref[2] = u * vec[:, 1:2]
    z_ref[3] = u * vec[:, 2:3]


def _tc_stage(sj, inner, rbf, cut, vec, w1, b1, w2, b2, wr, br):
    grid = (N_EDGES // _TCB,)
    return pl.pallas_call(
        _tc_body,
        grid=grid,
        in_specs=[
            pl.BlockSpec((_TCB, H), lambda e: (e, 0)),
            pl.BlockSpec((_TCB, H), lambda e: (e, 0)),
            pl.BlockSpec((_TCB, NUM_RBF), lambda e: (e, 0)),
            pl.BlockSpec((_TCB, 1), lambda e: (e, 0)),
            pl.BlockSpec((_TCB, 3), lambda e: (e, 0)),
            pl.BlockSpec((H, H), lambda e: (0, 0)),
            pl.BlockSpec((1, H), lambda e: (0, 0)),
            pl.BlockSpec((H, 3 * H), lambda e: (0, 0)),
            pl.BlockSpec((1, 3 * H), lambda e: (0, 0)),
            pl.BlockSpec((NUM_RBF, 3 * H), lambda e: (0, 0)),
            pl.BlockSpec((1, 3 * H), lambda e: (0, 0)),
        ],
        out_specs=pl.BlockSpec((4, _TCB, H), lambda e: (0, e, 0)),
        out_shape=jax.ShapeDtypeStruct((4, N_EDGES, H), jnp.float32),
    )(sj, inner, rbf, cut, vec, w1, b1, w2, b2, wr, br)


_SCC = 40
_SNCH = EPT // _SCC
_SNB = 4


@functools.cache
def _scatter_stage():
    @functools.partial(
        pl.kernel,
        out_type=jax.ShapeDtypeStruct((4, N_NODES, H), jnp.float32),
        mesh=_mesh(),
        scratch_types=[
            [pltpu.VMEM((_SCC,), jnp.int32)] * _SNB,
            [pltpu.VMEM((_SCC, H), jnp.float32)] * _SNB,
            pltpu.VMEM_SHARED((N_NODES, H), jnp.float32),
            [pltpu.SemaphoreType.DMA] * _SNB,
            [pltpu.SemaphoreType.DMA] * _SNB,
        ],
    )
    def scatter_k(i_hbm, z_hbm, zero_hbm, out4, idx_v, row_v, table, sem_ld,
                  sem_sc):
        core = lax.axis_index("c")
        sub = lax.axis_index("s")

        def zload(p, k, b):
            e0 = sub * EPT + k * _SCC
            return pltpu.async_copy(z_hbm.at[p, pl.ds(e0, _SCC), :], row_v[b],
                                    sem_ld[b])

        def wait_zload(p, k, b):
            e0 = sub * EPT + k * _SCC
            pltpu.make_async_copy(z_hbm.at[p, pl.ds(e0, _SCC), :], row_v[b],
                                  sem_ld[b]).wait()

        def wait_scatter(b):
            pltpu.make_async_copy(row_v[b], table.at[idx_v[b]],
                                  sem_sc[b]).wait()

        for q in range(2):
            p = 2 * core + q

            @pl.when(sub == 0)
            def _zero():
                pltpu.sync_copy(zero_hbm, table)

            plsc.subcore_barrier()

            zload(p, 0, 0)
            zload(p, 1, 1)

            def group(g, carry):
                for b in range(_SNB):
                    k = _SNB * g + b
                    e0 = sub * EPT + k * _SCC
                    wait_zload(p, k, b)
                    pltpu.sync_copy(i_hbm.at[pl.ds(e0, _SCC)], idx_v[b])
                    pltpu.async_copy(row_v[b], table.at[idx_v[b]], sem_sc[b],
                                     add=True)

                    @pl.when(k >= 2)
                    def _():
                        wait_scatter((b + 2) % _SNB)

                    @pl.when(k + 2 < _SNCH)
                    def _():
                        zload(p, k + 2, (b + 2) % _SNB)
                return carry

            lax.fori_loop(0, _SNCH // _SNB, group, 0)
            wait_scatter((_SNCH - 2) % _SNB)
            wait_scatter((_SNCH - 1) % _SNB)
            plsc.subcore_barrier()

            @pl.when(sub == 0)
            def _flush():
                pltpu.sync_copy(table, out4.at[p])

            plsc.subcore_barrier()

    return scatter_k


def kernel(s, v, edge_index, edge_rbf, edge_cutoff, edge_vec, W1, b1, W2, b2,
           Wr, br):
    i = edge_index[0].astype(jnp.int32)
    j = edge_index[1].astype(jnp.int32)
    n = s.shape[0]
    v2d = v.reshape(n, 3 * H)
    vecf = edge_vec.reshape(-1)

    sj, inner = _gather_stage()(j, s, v2d, vecf)
    z = _tc_stage(sj, inner, edge_rbf, edge_cutoff[:, None], edge_vec,
                  W1, b1[None, :], W2, b2[None, :], Wr, br[None, :])
    zero = jnp.zeros((n, H), jnp.float32)
    out4 = _scatter_stage()(i, z, zero)
    ds = out4[0]
    dv = jnp.transpose(out4[1:4], (1, 0, 2))
    return ds, dv


# speedup vs baseline: 1.1822x; 1.0510x over previous
import functools

import jax
import jax.numpy as jnp
from jax import lax
from jax.experimental import pallas as pl
from jax.experimental.pallas import tpu as pltpu
from jax.experimental.pallas import tpu_sc as plsc

N_NODES = 10000
N_EDGES = 320000
H = 128
NUM_RBF = 20

NC, NS = 2, 16
NW = NC * NS
EPW = N_EDGES // NW
EPT = N_EDGES // NS
GC = 80
SC_CHUNK = 80

def _mesh():
    return plsc.VectorSubcoreMesh(
        core_axis_name="c", subcore_axis_name="s", num_cores=NC, num_subcores=NS)


_NCH = EPW // GC


@functools.cache
def _gather_stage():
    @functools.partial(
        pl.kernel,
        out_type=[
            jax.ShapeDtypeStruct((N_EDGES, H), jnp.float32),
            jax.ShapeDtypeStruct((N_EDGES, H), jnp.float32),
        ],
        mesh=_mesh(),
        compiler_params=pltpu.CompilerParams(needs_layout_passes=False),
        scratch_types=[
            [pltpu.VMEM((GC,), jnp.int32)] * 2,
            [pltpu.VMEM((GC, H), jnp.float32)] * 2,
            [pltpu.VMEM((GC, 3 * H), jnp.float32)] * 2,
            [pltpu.VMEM((3 * GC,), jnp.float32)] * 2,
            pltpu.VMEM((GC, H), jnp.float32),
            [pltpu.SemaphoreType.DMA] * 2,
            [pltpu.SemaphoreType.DMA] * 2,
            [pltpu.SemaphoreType.DMA] * 2,
        ],
    )
    def gather_k(j_hbm, s_hbm, v_hbm, vecf_hbm, sj_out, inner_out, idx_v,
                 srow_v, vrow_v, vecc_v, ibuf_v, sem_s, sem_v, sem_c):
        wid = lax.axis_index("s") * NC + lax.axis_index("c")
        base = wid * EPW

        def issue(k, b):
            e0 = base + k * GC
            pltpu.sync_copy(j_hbm.at[pl.ds(e0, GC)], idx_v[b])
            pltpu.async_copy(s_hbm.at[idx_v[b]], srow_v[b], sem_s[b])
            pltpu.async_copy(v_hbm.at[idx_v[b]], vrow_v[b], sem_v[b])
            pltpu.async_copy(vecf_hbm.at[pl.ds(3 * e0, 3 * GC)], vecc_v[b],
                             sem_c[b])

        def consume(k, b):
            e0 = base + k * GC
            pltpu.make_async_copy(s_hbm.at[idx_v[b]], srow_v[b],
                                  sem_s[b]).wait()
            pltpu.sync_copy(srow_v[b], sj_out.at[pl.ds(e0, GC), :])
            pltpu.make_async_copy(v_hbm.at[idx_v[b]], vrow_v[b],
                                  sem_v[b]).wait()
            pltpu.make_async_copy(vecf_hbm.at[pl.ds(3 * e0, 3 * GC)],
                                  vecc_v[b], sem_c[b]).wait()

            def edge_body(e, carry):
                cs = [
                    plsc.load_gather(
                        vecc_v[b],
                        [jnp.full((16,), 3 * e + d, jnp.int32)])
                    for d in range(3)
                ]
                for kk in range(H // 16):
                    acc = (vrow_v[b][e, pl.ds(kk * 16, 16)] * cs[0]
                           + vrow_v[b][e, pl.ds(H + kk * 16, 16)] * cs[1]
                           + vrow_v[b][e, pl.ds(2 * H + kk * 16, 16)] * cs[2])
                    ibuf_v[e, pl.ds(kk * 16, 16)] = acc
                return carry

            lax.fori_loop(0, GC, edge_body, 0)
            pltpu.sync_copy(ibuf_v, inner_out.at[pl.ds(e0, GC), :])

        issue(0, 0)

        def body(m, carry):
            k0 = 2 * m

            @pl.when(k0 + 1 < _NCH)
            def _():
                issue(k0 + 1, 1)

            consume(k0, 0)

            @pl.when(k0 + 2 < _NCH)
            def _():
                issue(k0 + 2, 0)

            @pl.when(k0 + 1 < _NCH)
            def _():
                consume(k0 + 1, 1)

            return carry

        lax.fori_loop(0, (_NCH + 1) // 2, body, 0)

    return gather_k


_TCB = 640


def _tc_body(sj_ref, in_ref, rbf_ref, cut_ref, w1_ref, b1_ref,
             w2_ref, b2_ref, wr_ref, br_ref, z_ref):
    sj = sj_ref[...]
    h = jnp.dot(sj, w1_ref[...], preferred_element_type=jnp.float32) + b1_ref[...]
    h = h * (1.0 / (1.0 + jnp.exp(-h)))
    h = jnp.dot(h, w2_ref[...], preferred_element_type=jnp.float32) + b2_ref[...]
    wt = jnp.dot(rbf_ref[...], wr_ref[...], preferred_element_type=jnp.float32)
    wt = (wt + br_ref[...]) * cut_ref[...]
    x = h * wt
    x_ss = x[:, :H]
    x_sv = x[:, H:2 * H]
    x_vv = x[:, 2 * H:]
    u = x_sv + in_ref[...] * x_vv
    z_ref[0] = x_ss
    z_ref[1] = u


def _tc_stage(sj, inner, rbf, cut, w1, b1, w2, b2, wr, br):
    grid = (N_EDGES // _TCB,)
    return pl.pallas_call(
        _tc_body,
        grid=grid,
        in_specs=[
            pl.BlockSpec((_TCB, H), lambda e: (e, 0)),
            pl.BlockSpec((_TCB, H), lambda e: (e, 0)),
            pl.BlockSpec((_TCB, NUM_RBF), lambda e: (e, 0)),
            pl.BlockSpec((_TCB, 1), lambda e: (e, 0)),
            pl.BlockSpec((H, H), lambda e: (0, 0)),
            pl.BlockSpec((1, H), lambda e: (0, 0)),
            pl.BlockSpec((H, 3 * H), lambda e: (0, 0)),
            pl.BlockSpec((1, 3 * H), lambda e: (0, 0)),
            pl.BlockSpec((NUM_RBF, 3 * H), lambda e: (0, 0)),
            pl.BlockSpec((1, 3 * H), lambda e: (0, 0)),
        ],
        out_specs=pl.BlockSpec((2, _TCB, H), lambda e: (0, e, 0)),
        out_shape=jax.ShapeDtypeStruct((2, N_EDGES, H), jnp.float32),
    )(sj, inner, rbf, cut, w1, b1, w2, b2, wr, br)


_SCC = 40
_SNCH = EPT // _SCC
_SNB = 4


@functools.cache
def _scatter_stage():
    @functools.partial(
        pl.kernel,
        out_type=jax.ShapeDtypeStruct((4, N_NODES, H), jnp.float32),
        mesh=_mesh(),
        compiler_params=pltpu.CompilerParams(needs_layout_passes=False),
        scratch_types=[
            [pltpu.VMEM((_SCC,), jnp.int32)] * _SNB,
            [pltpu.VMEM((_SCC, H), jnp.float32)] * _SNB,
            [pltpu.VMEM((3 * _SCC,), jnp.float32)] * _SNB,
            pltpu.VMEM_SHARED((N_NODES, H), jnp.float32),
            [pltpu.SemaphoreType.DMA] * _SNB,
            [pltpu.SemaphoreType.DMA] * _SNB,
            [pltpu.SemaphoreType.DMA] * _SNB,
            [pltpu.SemaphoreType.DMA] * _SNB,
        ],
    )
    def scatter_k(i_hbm, z_hbm, vecf_hbm, zero_hbm, out4, idx_v, row_v, vec_v,
                  table, sem_ld, sem_sc, sem_ix, sem_vc):
        core = lax.axis_index("c")
        sub = lax.axis_index("s")

        def issue_loads(p, zsel, k, b):
            e0 = sub * EPT + k * _SCC
            pltpu.async_copy(z_hbm.at[zsel, pl.ds(e0, _SCC), :], row_v[b],
                             sem_ld[b])
            pltpu.async_copy(i_hbm.at[pl.ds(e0, _SCC)], idx_v[b], sem_ix[b])
            pltpu.async_copy(vecf_hbm.at[pl.ds(3 * e0, 3 * _SCC)], vec_v[b],
                             sem_vc[b])

        def wait_loads(p, zsel, k, b):
            e0 = sub * EPT + k * _SCC
            pltpu.make_async_copy(z_hbm.at[zsel, pl.ds(e0, _SCC), :],
                                  row_v[b], sem_ld[b]).wait()
            pltpu.make_async_copy(i_hbm.at[pl.ds(e0, _SCC)], idx_v[b],
                                  sem_ix[b]).wait()
            pltpu.make_async_copy(vecf_hbm.at[pl.ds(3 * e0, 3 * _SCC)],
                                  vec_v[b], sem_vc[b]).wait()

        def wait_scatter(b):
            pltpu.make_async_copy(row_v[b], table.at[idx_v[b]],
                                  sem_sc[b]).wait()

        for q in range(2):
            p = 2 * core + q
            zsel = jnp.minimum(p, 1)
            d = jnp.maximum(p - 1, 0)

            @pl.when(sub == 0)
            def _zero():
                pltpu.sync_copy(zero_hbm, table)

            plsc.subcore_barrier()

            issue_loads(p, zsel, 0, 0)
            issue_loads(p, zsel, 1, 1)

            def group(g, carry):
                for b in range(_SNB):
                    k = _SNB * g + b
                    wait_loads(p, zsel, k, b)

                    @pl.when(p > 0)
                    def _scale():
                        def edge_body(e, carry2):
                            c = plsc.load_gather(
                                vec_v[b],
                                [jnp.full((16,), 3 * e, jnp.int32) + d])
                            for kk in range(H // 16):
                                row_v[b][e, pl.ds(kk * 16, 16)] = (
                                    row_v[b][e, pl.ds(kk * 16, 16)] * c)
                            return carry2

                        lax.fori_loop(0, _SCC, edge_body, 0)

                    pltpu.async_copy(row_v[b], table.at[idx_v[b]], sem_sc[b],
                                     add=True)

                    @pl.when(k >= 2)
                    def _():
                        wait_scatter((b + 2) % _SNB)

                    @pl.when(k + 2 < _SNCH)
                    def _():
                        issue_loads(p, zsel, k + 2, (b + 2) % _SNB)
                return carry

            lax.fori_loop(0, _SNCH // _SNB, group, 0)
            wait_scatter((_SNCH - 2) % _SNB)
            wait_scatter((_SNCH - 1) % _SNB)
            plsc.subcore_barrier()

            @pl.when(sub == 0)
            def _flush():
                pltpu.sync_copy(table, out4.at[p])

            plsc.subcore_barrier()

    return scatter_k


def kernel(s, v, edge_index, edge_rbf, edge_cutoff, edge_vec, W1, b1, W2, b2,
           Wr, br):
    i = edge_index[0].astype(jnp.int32)
    j = edge_index[1].astype(jnp.int32)
    n = s.shape[0]
    v2d = v.reshape(n, 3 * H)
    vecf = edge_vec.reshape(-1)

    sj, inner = _gather_stage()(j, s, v2d, vecf)
    z = _tc_stage(sj, inner, edge_rbf, edge_cutoff[:, None],
                  W1, b1[None, :], W2, b2[None, :], Wr, br[None, :])
    zero = jnp.zeros((n, H), jnp.float32)
    out4 = _scatter_stage()(i, z, vecf, zero)
    ds = out4[0]
    dv = jnp.transpose(out4[1:4], (1, 0, 2))
    return ds, dv


# speedup vs baseline: 1.2767x; 1.0799x over previous
import functools

import jax
import jax.numpy as jnp
from jax import lax
from jax.experimental import pallas as pl
from jax.experimental.pallas import tpu as pltpu
from jax.experimental.pallas import tpu_sc as plsc

N_NODES = 10000
N_EDGES = 320000
H = 128
NUM_RBF = 20

NC, NS = 2, 16
NW = NC * NS
EPW = N_EDGES // NW
EPT = N_EDGES // NS
GC = 80
SC_CHUNK = 80

def _mesh():
    return plsc.VectorSubcoreMesh(
        core_axis_name="c", subcore_axis_name="s", num_cores=NC, num_subcores=NS)


_NCH = EPW // GC


@functools.cache
def _gather_stage():
    @functools.partial(
        pl.kernel,
        out_type=[
            jax.ShapeDtypeStruct((N_EDGES, H), jnp.float32),
            jax.ShapeDtypeStruct((N_EDGES, H), jnp.float32),
        ],
        mesh=_mesh(),
        compiler_params=pltpu.CompilerParams(needs_layout_passes=False),
        scratch_types=[
            [pltpu.VMEM((GC,), jnp.int32)] * 2,
            [pltpu.VMEM((GC, H), jnp.float32)] * 2,
            [pltpu.VMEM((GC, 3 * H), jnp.float32)] * 2,
            [pltpu.VMEM((3 * GC,), jnp.float32)] * 2,
            pltpu.VMEM((GC, H), jnp.float32),
            [pltpu.SemaphoreType.DMA] * 2,
            [pltpu.SemaphoreType.DMA] * 2,
            [pltpu.SemaphoreType.DMA] * 2,
        ],
    )
    def gather_k(j_hbm, s_hbm, v_hbm, vecf_hbm, sj_out, inner_out, idx_v,
                 srow_v, vrow_v, vecc_v, ibuf_v, sem_s, sem_v, sem_c):
        wid = lax.axis_index("s") * NC + lax.axis_index("c")
        base = wid * EPW

        def issue(k, b):
            e0 = base + k * GC
            pltpu.sync_copy(j_hbm.at[pl.ds(e0, GC)], idx_v[b])
            pltpu.async_copy(s_hbm.at[idx_v[b]], srow_v[b], sem_s[b])
            pltpu.async_copy(v_hbm.at[idx_v[b]], vrow_v[b], sem_v[b])
            pltpu.async_copy(vecf_hbm.at[pl.ds(3 * e0, 3 * GC)], vecc_v[b],
                             sem_c[b])

        def consume(k, b):
            e0 = base + k * GC
            pltpu.make_async_copy(s_hbm.at[idx_v[b]], srow_v[b],
                                  sem_s[b]).wait()
            pltpu.sync_copy(srow_v[b], sj_out.at[pl.ds(e0, GC), :])
            pltpu.make_async_copy(v_hbm.at[idx_v[b]], vrow_v[b],
                                  sem_v[b]).wait()
            pltpu.make_async_copy(vecf_hbm.at[pl.ds(3 * e0, 3 * GC)],
                                  vecc_v[b], sem_c[b]).wait()

            def edge_body(e, carry):
                cs = [
                    plsc.load_gather(
                        vecc_v[b],
                        [jnp.full((16,), 3 * e + d, jnp.int32)])
                    for d in range(3)
                ]
                for kk in range(H // 16):
                    acc = (vrow_v[b][e, pl.ds(kk * 16, 16)] * cs[0]
                           + vrow_v[b][e, pl.ds(H + kk * 16, 16)] * cs[1]
                           + vrow_v[b][e, pl.ds(2 * H + kk * 16, 16)] * cs[2])
                    ibuf_v[e, pl.ds(kk * 16, 16)] = acc
                return carry

            lax.fori_loop(0, GC, edge_body, 0)
            pltpu.sync_copy(ibuf_v, inner_out.at[pl.ds(e0, GC), :])

        issue(0, 0)

        def body(m, carry):
            k0 = 2 * m

            @pl.when(k0 + 1 < _NCH)
            def _():
                issue(k0 + 1, 1)

            consume(k0, 0)

            @pl.when(k0 + 2 < _NCH)
            def _():
                issue(k0 + 2, 0)

            @pl.when(k0 + 1 < _NCH)
            def _():
                consume(k0 + 1, 1)

            return carry

        lax.fori_loop(0, (_NCH + 1) // 2, body, 0)

    return gather_k


_TCB = 1280


def _tc_body(sj_ref, in_ref, rbf_ref, cut_ref, w1_ref, b1_ref,
             w2_ref, b2_ref, wr_ref, br_ref, z_ref):
    sj = sj_ref[...]
    h = jnp.dot(sj, w1_ref[...], preferred_element_type=jnp.float32) + b1_ref[...]
    h = h * (1.0 / (1.0 + jnp.exp(-h)))
    h = jnp.dot(h, w2_ref[...], preferred_element_type=jnp.float32) + b2_ref[...]
    wt = jnp.dot(rbf_ref[...], wr_ref[...], preferred_element_type=jnp.float32)
    wt = (wt + br_ref[...]) * cut_ref[...]
    x = h * wt
    x_ss = x[:, :H]
    x_sv = x[:, H:2 * H]
    x_vv = x[:, 2 * H:]
    u = x_sv + in_ref[...] * x_vv
    z_ref[0] = x_ss
    z_ref[1] = u


def _tc_stage(sj, inner, rbf, cut, w1, b1, w2, b2, wr, br):
    grid = (N_EDGES // _TCB,)
    return pl.pallas_call(
        _tc_body,
        grid=grid,
        in_specs=[
            pl.BlockSpec((_TCB, H), lambda e: (e, 0)),
            pl.BlockSpec((_TCB, H), lambda e: (e, 0)),
            pl.BlockSpec((_TCB, NUM_RBF), lambda e: (e, 0)),
            pl.BlockSpec((_TCB, 1), lambda e: (e, 0)),
            pl.BlockSpec((H, H), lambda e: (0, 0)),
            pl.BlockSpec((1, H), lambda e: (0, 0)),
            pl.BlockSpec((H, 3 * H), lambda e: (0, 0)),
            pl.BlockSpec((1, 3 * H), lambda e: (0, 0)),
            pl.BlockSpec((NUM_RBF, 3 * H), lambda e: (0, 0)),
            pl.BlockSpec((1, 3 * H), lambda e: (0, 0)),
        ],
        out_specs=pl.BlockSpec((2, _TCB, H), lambda e: (0, e, 0)),
        out_shape=jax.ShapeDtypeStruct((2, N_EDGES, H), jnp.float32),
    )(sj, inner, rbf, cut, w1, b1, w2, b2, wr, br)


_SCC = 40
_SNCH = EPT // _SCC
_SNB = 4


@functools.cache
def _scatter_stage():
    @functools.partial(
        pl.kernel,
        out_type=jax.ShapeDtypeStruct((4, N_NODES, H), jnp.float32),
        mesh=_mesh(),
        compiler_params=pltpu.CompilerParams(needs_layout_passes=False),
        scratch_types=[
            [pltpu.VMEM((_SCC,), jnp.int32)] * _SNB,
            [pltpu.VMEM((_SCC, H), jnp.float32)] * _SNB,
            [pltpu.VMEM((3 * _SCC,), jnp.float32)] * _SNB,
            pltpu.VMEM_SHARED((N_NODES, H), jnp.float32),
            [pltpu.SemaphoreType.DMA] * _SNB,
            [pltpu.SemaphoreType.DMA] * _SNB,
            [pltpu.SemaphoreType.DMA] * _SNB,
            [pltpu.SemaphoreType.DMA] * _SNB,
        ],
    )
    def scatter_k(i_hbm, z_hbm, vecf_hbm, zero_hbm, out4, idx_v, row_v, vec_v,
                  table, sem_ld, sem_sc, sem_ix, sem_vc):
        core = lax.axis_index("c")
        sub = lax.axis_index("s")

        def issue_loads(p, zsel, k, b):
            e0 = sub * EPT + k * _SCC
            pltpu.async_copy(z_hbm.at[zsel, pl.ds(e0, _SCC), :], row_v[b],
                             sem_ld[b])
            pltpu.async_copy(i_hbm.at[pl.ds(e0, _SCC)], idx_v[b], sem_ix[b])
            pltpu.async_copy(vecf_hbm.at[pl.ds(3 * e0, 3 * _SCC)], vec_v[b],
                             sem_vc[b])

        def wait_loads(p, zsel, k, b):
            e0 = sub * EPT + k * _SCC
            pltpu.make_async_copy(z_hbm.at[zsel, pl.ds(e0, _SCC), :],
                                  row_v[b], sem_ld[b]).wait()
            pltpu.make_async_copy(i_hbm.at[pl.ds(e0, _SCC)], idx_v[b],
                                  sem_ix[b]).wait()
            pltpu.make_async_copy(vecf_hbm.at[pl.ds(3 * e0, 3 * _SCC)],
                                  vec_v[b], sem_vc[b]).wait()

        def wait_scatter(b):
            pltpu.make_async_copy(row_v[b], table.at[idx_v[b]],
                                  sem_sc[b]).wait()

        for q in range(2):
            p = 2 * core + q
            zsel = jnp.minimum(p, 1)
            d = jnp.maximum(p - 1, 0)

            @pl.when(sub == 0)
            def _zero():
                pltpu.sync_copy(zero_hbm, table)

            plsc.subcore_barrier()

            issue_loads(p, zsel, 0, 0)
            issue_loads(p, zsel, 1, 1)

            def group(g, carry):
                for b in range(_SNB):
                    k = _SNB * g + b
                    wait_loads(p, zsel, k, b)

                    @pl.when(p > 0)
                    def _scale():
                        def edge_body(e, carry2):
                            c = plsc.load_gather(
                                vec_v[b],
                                [jnp.full((16,), 3 * e, jnp.int32) + d])
                            for kk in range(H // 16):
                                row_v[b][e, pl.ds(kk * 16, 16)] = (
                                    row_v[b][e, pl.ds(kk * 16, 16)] * c)
                            return carry2

                        lax.fori_loop(0, _SCC, edge_body, 0)

                    pltpu.async_copy(row_v[b], table.at[idx_v[b]], sem_sc[b],
                                     add=True)

                    @pl.when(k >= 2)
                    def _():
                        wait_scatter((b + 2) % _SNB)

                    @pl.when(k + 2 < _SNCH)
                    def _():
                        issue_loads(p, zsel, k + 2, (b + 2) % _SNB)
                return carry

            lax.fori_loop(0, _SNCH // _SNB, group, 0)
            wait_scatter((_SNCH - 2) % _SNB)
            wait_scatter((_SNCH - 1) % _SNB)
            plsc.subcore_barrier()

            @pl.when(sub == 0)
            def _flush():
                pltpu.sync_copy(table, out4.at[p])

            plsc.subcore_barrier()

    return scatter_k


def kernel(s, v, edge_index, edge_rbf, edge_cutoff, edge_vec, W1, b1, W2, b2,
           Wr, br):
    i = edge_index[0].astype(jnp.int32)
    j = edge_index[1].astype(jnp.int32)
    n = s.shape[0]
    v2d = v.reshape(n, 3 * H)
    vecf = edge_vec.reshape(-1)

    sj, inner = _gather_stage()(j, s, v2d, vecf)
    z = _tc_stage(sj, inner, edge_rbf, edge_cutoff[:, None],
                  W1, b1[None, :], W2, b2[None, :], Wr, br[None, :])
    zero = jnp.zeros((n, H), jnp.float32)
    out4 = _scatter_stage()(i, z, vecf, zero)
    ds = out4[0]
    dv = jnp.transpose(out4[1:4], (1, 0, 2))
    return ds, dv


# speedup vs baseline: 1.3285x; 1.0405x over previous
import functools

import jax
import jax.numpy as jnp
from jax import lax
from jax.experimental import pallas as pl
from jax.experimental.pallas import tpu as pltpu
from jax.experimental.pallas import tpu_sc as plsc

N_NODES = 10000
N_EDGES = 320000
H = 128
NUM_RBF = 20

NC, NS = 2, 16
NW = NC * NS
EPW = N_EDGES // NW
EPT = N_EDGES // NS
GC = 80
SC_CHUNK = 80

def _mesh():
    return plsc.VectorSubcoreMesh(
        core_axis_name="c", subcore_axis_name="s", num_cores=NC, num_subcores=NS)


_NCH = EPW // GC


@functools.cache
def _gather_stage():
    @functools.partial(
        pl.kernel,
        out_type=[
            jax.ShapeDtypeStruct((N_EDGES, H), jnp.float32),
            jax.ShapeDtypeStruct((N_EDGES, H), jnp.float32),
        ],
        mesh=_mesh(),
        compiler_params=pltpu.CompilerParams(needs_layout_passes=False),
        scratch_types=[
            [pltpu.VMEM((GC,), jnp.int32)] * 2,
            [pltpu.VMEM((GC, H), jnp.float32)] * 2,
            [pltpu.VMEM((GC, 3 * H), jnp.float32)] * 2,
            [pltpu.VMEM((3 * GC,), jnp.float32)] * 2,
            pltpu.VMEM((GC, H), jnp.float32),
            [pltpu.SemaphoreType.DMA] * 2,
            [pltpu.SemaphoreType.DMA] * 2,
            [pltpu.SemaphoreType.DMA] * 2,
        ],
    )
    def gather_k(j_hbm, s_hbm, v_hbm, vecf_hbm, sj_out, inner_out, idx_v,
                 srow_v, vrow_v, vecc_v, ibuf_v, sem_s, sem_v, sem_c):
        wid = lax.axis_index("s") * NC + lax.axis_index("c")
        base = wid * EPW

        def issue(k, b):
            e0 = base + k * GC
            pltpu.sync_copy(j_hbm.at[pl.ds(e0, GC)], idx_v[b])
            pltpu.async_copy(s_hbm.at[idx_v[b]], srow_v[b], sem_s[b])
            pltpu.async_copy(v_hbm.at[idx_v[b]], vrow_v[b], sem_v[b])
            pltpu.async_copy(vecf_hbm.at[pl.ds(3 * e0, 3 * GC)], vecc_v[b],
                             sem_c[b])

        def consume(k, b):
            e0 = base + k * GC
            pltpu.make_async_copy(s_hbm.at[idx_v[b]], srow_v[b],
                                  sem_s[b]).wait()
            pltpu.sync_copy(srow_v[b], sj_out.at[pl.ds(e0, GC), :])
            pltpu.make_async_copy(v_hbm.at[idx_v[b]], vrow_v[b],
                                  sem_v[b]).wait()
            pltpu.make_async_copy(vecf_hbm.at[pl.ds(3 * e0, 3 * GC)],
                                  vecc_v[b], sem_c[b]).wait()

            def edge_body(e, carry):
                cs = [
                    plsc.load_gather(
                        vecc_v[b],
                        [jnp.full((16,), 3 * e + d, jnp.int32)])
                    for d in range(3)
                ]
                for kk in range(H // 16):
                    acc = (vrow_v[b][e, pl.ds(kk * 16, 16)] * cs[0]
                           + vrow_v[b][e, pl.ds(H + kk * 16, 16)] * cs[1]
                           + vrow_v[b][e, pl.ds(2 * H + kk * 16, 16)] * cs[2])
                    ibuf_v[e, pl.ds(kk * 16, 16)] = acc
                return carry

            lax.fori_loop(0, GC, edge_body, 0)
            pltpu.sync_copy(ibuf_v, inner_out.at[pl.ds(e0, GC), :])

        issue(0, 0)

        def body(m, carry):
            k0 = 2 * m

            @pl.when(k0 + 1 < _NCH)
            def _():
                issue(k0 + 1, 1)

            consume(k0, 0)

            @pl.when(k0 + 2 < _NCH)
            def _():
                issue(k0 + 2, 0)

            @pl.when(k0 + 1 < _NCH)
            def _():
                consume(k0 + 1, 1)

            return carry

        lax.fori_loop(0, (_NCH + 1) // 2, body, 0)

    return gather_k


_TCB = 2560


def _tc_body(sj_ref, in_ref, rbf_ref, cut_ref, w1_ref, b1_ref,
             w2_ref, b2_ref, wr_ref, br_ref, z_ref):
    sj = sj_ref[...]
    h = jnp.dot(sj, w1_ref[...], preferred_element_type=jnp.float32) + b1_ref[...]
    h = h * (1.0 / (1.0 + jnp.exp(-h)))
    h = jnp.dot(h, w2_ref[...], preferred_element_type=jnp.float32) + b2_ref[...]
    wt = jnp.dot(rbf_ref[...], wr_ref[...], preferred_element_type=jnp.float32)
    wt = (wt + br_ref[...]) * cut_ref[...]
    x = h * wt
    x_ss = x[:, :H]
    x_sv = x[:, H:2 * H]
    x_vv = x[:, 2 * H:]
    u = x_sv + in_ref[...] * x_vv
    z_ref[0] = x_ss
    z_ref[1] = u


def _tc_stage(sj, inner, rbf, cut, w1, b1, w2, b2, wr, br):
    grid = (N_EDGES // _TCB,)
    return pl.pallas_call(
        _tc_body,
        grid=grid,
        in_specs=[
            pl.BlockSpec((_TCB, H), lambda e: (e, 0)),
            pl.BlockSpec((_TCB, H), lambda e: (e, 0)),
            pl.BlockSpec((_TCB, NUM_RBF), lambda e: (e, 0)),
            pl.BlockSpec((_TCB, 1), lambda e: (e, 0)),
            pl.BlockSpec((H, H), lambda e: (0, 0)),
            pl.BlockSpec((1, H), lambda e: (0, 0)),
            pl.BlockSpec((H, 3 * H), lambda e: (0, 0)),
            pl.BlockSpec((1, 3 * H), lambda e: (0, 0)),
            pl.BlockSpec((NUM_RBF, 3 * H), lambda e: (0, 0)),
            pl.BlockSpec((1, 3 * H), lambda e: (0, 0)),
        ],
        out_specs=pl.BlockSpec((2, _TCB, H), lambda e: (0, e, 0)),
        out_shape=jax.ShapeDtypeStruct((2, N_EDGES, H), jnp.float32),
    )(sj, inner, rbf, cut, w1, b1, w2, b2, wr, br)


_SCC = 40
_SNCH = EPT // _SCC
_SNB = 4


@functools.cache
def _scatter_stage():
    @functools.partial(
        pl.kernel,
        out_type=jax.ShapeDtypeStruct((4, N_NODES, H), jnp.float32),
        mesh=_mesh(),
        compiler_params=pltpu.CompilerParams(needs_layout_passes=False),
        scratch_types=[
            [pltpu.VMEM((_SCC,), jnp.int32)] * _SNB,
            [pltpu.VMEM((_SCC, H), jnp.float32)] * _SNB,
            [pltpu.VMEM((3 * _SCC,), jnp.float32)] * _SNB,
            pltpu.VMEM_SHARED((N_NODES, H), jnp.float32),
            [pltpu.SemaphoreType.DMA] * _SNB,
            [pltpu.SemaphoreType.DMA] * _SNB,
            [pltpu.SemaphoreType.DMA] * _SNB,
            [pltpu.SemaphoreType.DMA] * _SNB,
        ],
    )
    def scatter_k(i_hbm, z_hbm, vecf_hbm, zero_hbm, out4, idx_v, row_v, vec_v,
                  table, sem_ld, sem_sc, sem_ix, sem_vc):
        core = lax.axis_index("c")
        sub = lax.axis_index("s")

        def issue_loads(p, zsel, k, b):
            e0 = sub * EPT + k * _SCC
            pltpu.async_copy(z_hbm.at[zsel, pl.ds(e0, _SCC), :], row_v[b],
                             sem_ld[b])
            pltpu.async_copy(i_hbm.at[pl.ds(e0, _SCC)], idx_v[b], sem_ix[b])
            pltpu.async_copy(vecf_hbm.at[pl.ds(3 * e0, 3 * _SCC)], vec_v[b],
                             sem_vc[b])

        def wait_loads(p, zsel, k, b):
            e0 = sub * EPT + k * _SCC
            pltpu.make_async_copy(z_hbm.at[zsel, pl.ds(e0, _SCC), :],
                                  row_v[b], sem_ld[b]).wait()
            pltpu.make_async_copy(i_hbm.at[pl.ds(e0, _SCC)], idx_v[b],
                                  sem_ix[b]).wait()
            pltpu.make_async_copy(vecf_hbm.at[pl.ds(3 * e0, 3 * _SCC)],
                                  vec_v[b], sem_vc[b]).wait()

        def wait_scatter(b):
            pltpu.make_async_copy(row_v[b], table.at[idx_v[b]],
                                  sem_sc[b]).wait()

        for q in range(2):
            p = 2 * core + q
            zsel = jnp.minimum(p, 1)
            d = jnp.maximum(p - 1, 0)

            @pl.when(sub == 0)
            def _zero():
                pltpu.sync_copy(zero_hbm, table)

            plsc.subcore_barrier()

            issue_loads(p, zsel, 0, 0)
            issue_loads(p, zsel, 1, 1)

            def group(g, carry):
                for b in range(_SNB):
                    k = _SNB * g + b
                    wait_loads(p, zsel, k, b)

                    @pl.when(p > 0)
                    def _scale():
                        def edge_body(e, carry2):
                            c = plsc.load_gather(
                                vec_v[b],
                                [jnp.full((16,), 3 * e, jnp.int32) + d])
                            for kk in range(H // 16):
                                row_v[b][e, pl.ds(kk * 16, 16)] = (
                                    row_v[b][e, pl.ds(kk * 16, 16)] * c)
                            return carry2

                        lax.fori_loop(0, _SCC, edge_body, 0)

                    pltpu.async_copy(row_v[b], table.at[idx_v[b]], sem_sc[b],
                                     add=True)

                    @pl.when(k >= 2)
                    def _():
                        wait_scatter((b + 2) % _SNB)

                    @pl.when(k + 2 < _SNCH)
                    def _():
                        issue_loads(p, zsel, k + 2, (b + 2) % _SNB)
                return carry

            lax.fori_loop(0, _SNCH // _SNB, group, 0)
            wait_scatter((_SNCH - 2) % _SNB)
            wait_scatter((_SNCH - 1) % _SNB)
            plsc.subcore_barrier()

            @pl.when(sub == 0)
            def _flush():
                pltpu.sync_copy(table, out4.at[p])

            plsc.subcore_barrier()

    return scatter_k


def kernel(s, v, edge_index, edge_rbf, edge_cutoff, edge_vec, W1, b1, W2, b2,
           Wr, br):
    i = edge_index[0].astype(jnp.int32)
    j = edge_index[1].astype(jnp.int32)
    n = s.shape[0]
    v2d = v.reshape(n, 3 * H)
    vecf = edge_vec.reshape(-1)

    sj, inner = _gather_stage()(j, s, v2d, vecf)
    z = _tc_stage(sj, inner, edge_rbf, edge_cutoff[:, None],
                  W1, b1[None, :], W2, b2[None, :], Wr, br[None, :])
    zero = jnp.zeros((n, H), jnp.float32)
    out4 = _scatter_stage()(i, z, vecf, zero)
    ds = out4[0]
    dv = jnp.transpose(out4[1:4], (1, 0, 2))
    return ds, dv


# speedup vs baseline: 1.3356x; 1.0053x over previous
import functools

import jax
import jax.numpy as jnp
from jax import lax
from jax.experimental import pallas as pl
from jax.experimental.pallas import tpu as pltpu
from jax.experimental.pallas import tpu_sc as plsc

N_NODES = 10000
N_EDGES = 320000
H = 128
NUM_RBF = 20

NC, NS = 2, 16
NW = NC * NS
EPW = N_EDGES // NW
EPT = N_EDGES // NS
GC = 80
SC_CHUNK = 80

def _mesh():
    return plsc.VectorSubcoreMesh(
        core_axis_name="c", subcore_axis_name="s", num_cores=NC, num_subcores=NS)


_NCH = EPW // GC


@functools.cache
def _gather_stage():
    @functools.partial(
        pl.kernel,
        out_type=[
            jax.ShapeDtypeStruct((N_EDGES, H), jnp.float32),
            jax.ShapeDtypeStruct((N_EDGES, H), jnp.float32),
        ],
        mesh=_mesh(),
        compiler_params=pltpu.CompilerParams(needs_layout_passes=False),
        scratch_types=[
            [pltpu.VMEM((GC,), jnp.int32)] * 2,
            [pltpu.VMEM((GC, H), jnp.float32)] * 2,
            [pltpu.VMEM((GC, 3 * H), jnp.float32)] * 2,
            [pltpu.VMEM((3 * GC,), jnp.float32)] * 2,
            pltpu.VMEM((GC, H), jnp.float32),
            [pltpu.SemaphoreType.DMA] * 2,
            [pltpu.SemaphoreType.DMA] * 2,
            [pltpu.SemaphoreType.DMA] * 2,
        ],
    )
    def gather_k(j_hbm, s_hbm, v_hbm, vecf_hbm, sj_out, inner_out, idx_v,
                 srow_v, vrow_v, vecc_v, ibuf_v, sem_s, sem_v, sem_c):
        wid = lax.axis_index("s") * NC + lax.axis_index("c")
        base = wid * EPW

        def issue(k, b):
            e0 = base + k * GC
            pltpu.sync_copy(j_hbm.at[pl.ds(e0, GC)], idx_v[b])
            pltpu.async_copy(s_hbm.at[idx_v[b]], srow_v[b], sem_s[b])
            pltpu.async_copy(v_hbm.at[idx_v[b]], vrow_v[b], sem_v[b])
            pltpu.async_copy(vecf_hbm.at[pl.ds(3 * e0, 3 * GC)], vecc_v[b],
                             sem_c[b])

        def consume(k, b):
            e0 = base + k * GC
            pltpu.make_async_copy(s_hbm.at[idx_v[b]], srow_v[b],
                                  sem_s[b]).wait()
            pltpu.sync_copy(srow_v[b], sj_out.at[pl.ds(e0, GC), :])
            pltpu.make_async_copy(v_hbm.at[idx_v[b]], vrow_v[b],
                                  sem_v[b]).wait()
            pltpu.make_async_copy(vecf_hbm.at[pl.ds(3 * e0, 3 * GC)],
                                  vecc_v[b], sem_c[b]).wait()

            def edge_body(e, carry):
                cs = [
                    plsc.load_gather(
                        vecc_v[b],
                        [jnp.full((16,), 3 * e + d, jnp.int32)])
                    for d in range(3)
                ]
                for kk in range(H // 16):
                    acc = (vrow_v[b][e, pl.ds(kk * 16, 16)] * cs[0]
                           + vrow_v[b][e, pl.ds(H + kk * 16, 16)] * cs[1]
                           + vrow_v[b][e, pl.ds(2 * H + kk * 16, 16)] * cs[2])
                    ibuf_v[e, pl.ds(kk * 16, 16)] = acc
                return carry

            lax.fori_loop(0, GC, edge_body, 0)
            pltpu.sync_copy(ibuf_v, inner_out.at[pl.ds(e0, GC), :])

        issue(0, 0)

        def body(m, carry):
            k0 = 2 * m

            @pl.when(k0 + 1 < _NCH)
            def _():
                issue(k0 + 1, 1)

            consume(k0, 0)

            @pl.when(k0 + 2 < _NCH)
            def _():
                issue(k0 + 2, 0)

            @pl.when(k0 + 1 < _NCH)
            def _():
                consume(k0 + 1, 1)

            return carry

        lax.fori_loop(0, (_NCH + 1) // 2, body, 0)

    return gather_k


_TCB = 3200


def _tc_body(sj_ref, in_ref, rbf_ref, cut_ref, w1_ref, b1_ref,
             w2_ref, b2_ref, wr_ref, br_ref, z_ref):
    sj = sj_ref[...]
    h = jnp.dot(sj, w1_ref[...], preferred_element_type=jnp.float32) + b1_ref[...]
    h = h * (1.0 / (1.0 + jnp.exp(-h)))
    h = jnp.dot(h, w2_ref[...], preferred_element_type=jnp.float32) + b2_ref[...]
    wt = jnp.dot(rbf_ref[...], wr_ref[...], preferred_element_type=jnp.float32)
    wt = (wt + br_ref[...]) * cut_ref[...]
    x = h * wt
    x_ss = x[:, :H]
    x_sv = x[:, H:2 * H]
    x_vv = x[:, 2 * H:]
    u = x_sv + in_ref[...] * x_vv
    z_ref[0] = x_ss
    z_ref[1] = u


def _tc_stage(sj, inner, rbf, cut, w1, b1, w2, b2, wr, br):
    grid = (N_EDGES // _TCB,)
    return pl.pallas_call(
        _tc_body,
        grid=grid,
        in_specs=[
            pl.BlockSpec((_TCB, H), lambda e: (e, 0)),
            pl.BlockSpec((_TCB, H), lambda e: (e, 0)),
            pl.BlockSpec((_TCB, NUM_RBF), lambda e: (e, 0)),
            pl.BlockSpec((_TCB, 1), lambda e: (e, 0)),
            pl.BlockSpec((H, H), lambda e: (0, 0)),
            pl.BlockSpec((1, H), lambda e: (0, 0)),
            pl.BlockSpec((H, 3 * H), lambda e: (0, 0)),
            pl.BlockSpec((1, 3 * H), lambda e: (0, 0)),
            pl.BlockSpec((NUM_RBF, 3 * H), lambda e: (0, 0)),
            pl.BlockSpec((1, 3 * H), lambda e: (0, 0)),
        ],
        out_specs=pl.BlockSpec((2, _TCB, H), lambda e: (0, e, 0)),
        out_shape=jax.ShapeDtypeStruct((2, N_EDGES, H), jnp.float32),
    )(sj, inner, rbf, cut, w1, b1, w2, b2, wr, br)


_SCC = 40
_SNCH = EPT // _SCC
_SNB = 4


@functools.cache
def _scatter_stage():
    @functools.partial(
        pl.kernel,
        out_type=jax.ShapeDtypeStruct((4, N_NODES, H), jnp.float32),
        mesh=_mesh(),
        compiler_params=pltpu.CompilerParams(needs_layout_passes=False),
        scratch_types=[
            [pltpu.VMEM((_SCC,), jnp.int32)] * _SNB,
            [pltpu.VMEM((_SCC, H), jnp.float32)] * _SNB,
            [pltpu.VMEM((3 * _SCC,), jnp.float32)] * _SNB,
            pltpu.VMEM_SHARED((N_NODES, H), jnp.float32),
            [pltpu.SemaphoreType.DMA] * _SNB,
            [pltpu.SemaphoreType.DMA] * _SNB,
            [pltpu.SemaphoreType.DMA] * _SNB,
            [pltpu.SemaphoreType.DMA] * _SNB,
        ],
    )
    def scatter_k(i_hbm, z_hbm, vecf_hbm, zero_hbm, out4, idx_v, row_v, vec_v,
                  table, sem_ld, sem_sc, sem_ix, sem_vc):
        core = lax.axis_index("c")
        sub = lax.axis_index("s")

        def issue_loads(p, zsel, k, b):
            e0 = sub * EPT + k * _SCC
            pltpu.async_copy(z_hbm.at[zsel, pl.ds(e0, _SCC), :], row_v[b],
                             sem_ld[b])
            pltpu.async_copy(i_hbm.at[pl.ds(e0, _SCC)], idx_v[b], sem_ix[b])
            pltpu.async_copy(vecf_hbm.at[pl.ds(3 * e0, 3 * _SCC)], vec_v[b],
                             sem_vc[b])

        def wait_loads(p, zsel, k, b):
            e0 = sub * EPT + k * _SCC
            pltpu.make_async_copy(z_hbm.at[zsel, pl.ds(e0, _SCC), :],
                                  row_v[b], sem_ld[b]).wait()
            pltpu.make_async_copy(i_hbm.at[pl.ds(e0, _SCC)], idx_v[b],
                                  sem_ix[b]).wait()
            pltpu.make_async_copy(vecf_hbm.at[pl.ds(3 * e0, 3 * _SCC)],
                                  vec_v[b], sem_vc[b]).wait()

        def wait_scatter(b):
            pltpu.make_async_copy(row_v[b], table.at[idx_v[b]],
                                  sem_sc[b]).wait()

        for q in range(2):
            p = 2 * core + q
            zsel = jnp.minimum(p, 1)
            d = jnp.maximum(p - 1, 0)

            @pl.when(sub == 0)
            def _zero():
                pltpu.sync_copy(zero_hbm, table)

            plsc.subcore_barrier()

            issue_loads(p, zsel, 0, 0)
            issue_loads(p, zsel, 1, 1)

            def group(g, carry):
                for b in range(_SNB):
                    k = _SNB * g + b
                    wait_loads(p, zsel, k, b)

                    @pl.when(p > 0)
                    def _scale():
                        def edge_body(e, carry2):
                            c = plsc.load_gather(
                                vec_v[b],
                                [jnp.full((16,), 3 * e, jnp.int32) + d])
                            for kk in range(H // 16):
                                row_v[b][e, pl.ds(kk * 16, 16)] = (
                                    row_v[b][e, pl.ds(kk * 16, 16)] * c)
                            return carry2

                        lax.fori_loop(0, _SCC, edge_body, 0)

                    pltpu.async_copy(row_v[b], table.at[idx_v[b]], sem_sc[b],
                                     add=True)

                    @pl.when(k >= 2)
                    def _():
                        wait_scatter((b + 2) % _SNB)

                    @pl.when(k + 2 < _SNCH)
                    def _():
                        issue_loads(p, zsel, k + 2, (b + 2) % _SNB)
                return carry

            lax.fori_loop(0, _SNCH // _SNB, group, 0)
            wait_scatter((_SNCH - 2) % _SNB)
            wait_scatter((_SNCH - 1) % _SNB)
            plsc.subcore_barrier()

            @pl.when(sub == 0)
            def _flush():
                pltpu.sync_copy(table, out4.at[p])

            plsc.subcore_barrier()

    return scatter_k


def kernel(s, v, edge_index, edge_rbf, edge_cutoff, edge_vec, W1, b1, W2, b2,
           Wr, br):
    i = edge_index[0].astype(jnp.int32)
    j = edge_index[1].astype(jnp.int32)
    n = s.shape[0]
    v2d = v.reshape(n, 3 * H)
    vecf = edge_vec.reshape(-1)

    sj, inner = _gather_stage()(j, s, v2d, vecf)
    z = _tc_stage(sj, inner, edge_rbf, edge_cutoff[:, None],
                  W1, b1[None, :], W2, b2[None, :], Wr, br[None, :])
    zero = jnp.zeros((n, H), jnp.float32)
    out4 = _scatter_stage()(i, z, vecf, zero)
    ds = out4[0]
    dv = jnp.transpose(out4[1:4], (1, 0, 2))
    return ds, dv


# speedup vs baseline: 1.3694x; 1.0253x over previous
import functools

import jax
import jax.numpy as jnp
from jax import lax
from jax.experimental import pallas as pl
from jax.experimental.pallas import tpu as pltpu
from jax.experimental.pallas import tpu_sc as plsc

N_NODES = 10000
N_EDGES = 320000
H = 128
NUM_RBF = 20

NC, NS = 2, 16
NW = NC * NS
EPW = N_EDGES // NW
EPT = N_EDGES // NS
GC = 40
SC_CHUNK = 80

def _mesh():
    return plsc.VectorSubcoreMesh(
        core_axis_name="c", subcore_axis_name="s", num_cores=NC, num_subcores=NS)


_GNCH = EPW // GC
_GNB = 4


@functools.cache
def _gather_stage():
    @functools.partial(
        pl.kernel,
        out_type=[
            jax.ShapeDtypeStruct((N_EDGES, H), jnp.float32),
            jax.ShapeDtypeStruct((N_EDGES, H), jnp.float32),
        ],
        mesh=_mesh(),
        compiler_params=pltpu.CompilerParams(needs_layout_passes=False),
        scratch_types=[
            [pltpu.VMEM((GC,), jnp.int32)] * _GNB,
            [pltpu.VMEM((GC, H), jnp.float32)] * _GNB,
            [pltpu.VMEM((GC, 3 * H), jnp.float32)] * _GNB,
            [pltpu.VMEM((3 * GC,), jnp.float32)] * _GNB,
            [pltpu.VMEM((GC, H), jnp.float32)] * _GNB,
            [pltpu.SemaphoreType.DMA] * _GNB,
            [pltpu.SemaphoreType.DMA] * _GNB,
            [pltpu.SemaphoreType.DMA] * _GNB,
            [pltpu.SemaphoreType.DMA] * _GNB,
            [pltpu.SemaphoreType.DMA] * _GNB,
        ],
    )
    def gather_k(j_hbm, s_hbm, v_hbm, vecf_hbm, sj_out, inner_out, idx_v,
                 srow_v, vrow_v, vecc_v, ibuf_v, sem_s, sem_v, sem_c, sem_ws,
                 sem_wi):
        wid = lax.axis_index("s") * NC + lax.axis_index("c")
        base = wid * EPW

        def issue(k, b):
            e0 = base + k * GC
            pltpu.sync_copy(j_hbm.at[pl.ds(e0, GC)], idx_v[b])
            pltpu.async_copy(s_hbm.at[idx_v[b]], srow_v[b], sem_s[b])
            pltpu.async_copy(v_hbm.at[idx_v[b]], vrow_v[b], sem_v[b])
            pltpu.async_copy(vecf_hbm.at[pl.ds(3 * e0, 3 * GC)], vecc_v[b],
                             sem_c[b])

        def wait_wb(k, b):
            e0 = base + k * GC
            pltpu.make_async_copy(srow_v[b], sj_out.at[pl.ds(e0, GC), :],
                                  sem_ws[b]).wait()
            pltpu.make_async_copy(ibuf_v[b], inner_out.at[pl.ds(e0, GC), :],
                                  sem_wi[b]).wait()

        def consume(k, b):
            e0 = base + k * GC
            pltpu.make_async_copy(s_hbm.at[idx_v[b]], srow_v[b],
                                  sem_s[b]).wait()
            pltpu.async_copy(srow_v[b], sj_out.at[pl.ds(e0, GC), :],
                             sem_ws[b])
            pltpu.make_async_copy(v_hbm.at[idx_v[b]], vrow_v[b],
                                  sem_v[b]).wait()
            pltpu.make_async_copy(vecf_hbm.at[pl.ds(3 * e0, 3 * GC)],
                                  vecc_v[b], sem_c[b]).wait()

            def edge_body(e, carry):
                cs = [
                    plsc.load_gather(
                        vecc_v[b],
                        [jnp.full((16,), 3 * e + d, jnp.int32)])
                    for d in range(3)
                ]
                for kk in range(H // 16):
                    acc = (vrow_v[b][e, pl.ds(kk * 16, 16)] * cs[0]
                           + vrow_v[b][e, pl.ds(H + kk * 16, 16)] * cs[1]
                           + vrow_v[b][e, pl.ds(2 * H + kk * 16, 16)] * cs[2])
                    ibuf_v[b][e, pl.ds(kk * 16, 16)] = acc
                return carry

            lax.fori_loop(0, GC, edge_body, 0)
            pltpu.async_copy(ibuf_v[b], inner_out.at[pl.ds(e0, GC), :],
                             sem_wi[b])

        issue(0, 0)
        issue(1, 1)

        def group(g, carry):
            for b in range(_GNB):
                k = _GNB * g + b

                @pl.when(k < _GNCH)
                def _():
                    consume(k, b)

                    @pl.when(k >= 2)
                    def _():
                        wait_wb(k - 2, (b + 2) % _GNB)

                    @pl.when(k + 2 < _GNCH)
                    def _():
                        issue(k + 2, (b + 2) % _GNB)

            return carry

        lax.fori_loop(0, (_GNCH + _GNB - 1) // _GNB, group, 0)
        wait_wb(_GNCH - 2, (_GNCH - 2) % _GNB)
        wait_wb(_GNCH - 1, (_GNCH - 1) % _GNB)

    return gather_k


_TCB = 3200


def _tc_body(sj_ref, in_ref, rbf_ref, cut_ref, w1_ref, b1_ref,
             w2_ref, b2_ref, wr_ref, br_ref, z_ref):
    sj = sj_ref[...]
    h = jnp.dot(sj, w1_ref[...], preferred_element_type=jnp.float32) + b1_ref[...]
    h = h * (1.0 / (1.0 + jnp.exp(-h)))
    h = jnp.dot(h, w2_ref[...], preferred_element_type=jnp.float32) + b2_ref[...]
    wt = jnp.dot(rbf_ref[...], wr_ref[...], preferred_element_type=jnp.float32)
    wt = (wt + br_ref[...]) * cut_ref[...]
    x = h * wt
    x_ss = x[:, :H]
    x_sv = x[:, H:2 * H]
    x_vv = x[:, 2 * H:]
    u = x_sv + in_ref[...] * x_vv
    z_ref[0] = x_ss
    z_ref[1] = u


def _tc_stage(sj, inner, rbf, cut, w1, b1, w2, b2, wr, br):
    grid = (N_EDGES // _TCB,)
    return pl.pallas_call(
        _tc_body,
        grid=grid,
        in_specs=[
            pl.BlockSpec((_TCB, H), lambda e: (e, 0)),
            pl.BlockSpec((_TCB, H), lambda e: (e, 0)),
            pl.BlockSpec((_TCB, NUM_RBF), lambda e: (e, 0)),
            pl.BlockSpec((_TCB, 1), lambda e: (e, 0)),
            pl.BlockSpec((H, H), lambda e: (0, 0)),
            pl.BlockSpec((1, H), lambda e: (0, 0)),
            pl.BlockSpec((H, 3 * H), lambda e: (0, 0)),
            pl.BlockSpec((1, 3 * H), lambda e: (0, 0)),
            pl.BlockSpec((NUM_RBF, 3 * H), lambda e: (0, 0)),
            pl.BlockSpec((1, 3 * H), lambda e: (0, 0)),
        ],
        out_specs=pl.BlockSpec((2, _TCB, H), lambda e: (0, e, 0)),
        out_shape=jax.ShapeDtypeStruct((2, N_EDGES, H), jnp.float32),
    )(sj, inner, rbf, cut, w1, b1, w2, b2, wr, br)


_SCC = 40
_SNCH = EPT // _SCC
_SNB = 4


@functools.cache
def _scatter_stage():
    @functools.partial(
        pl.kernel,
        out_type=jax.ShapeDtypeStruct((4, N_NODES, H), jnp.float32),
        mesh=_mesh(),
        compiler_params=pltpu.CompilerParams(needs_layout_passes=False),
        scratch_types=[
            [pltpu.VMEM((_SCC,), jnp.int32)] * _SNB,
            [pltpu.VMEM((_SCC, H), jnp.float32)] * _SNB,
            [pltpu.VMEM((3 * _SCC,), jnp.float32)] * _SNB,
            pltpu.VMEM_SHARED((N_NODES, H), jnp.float32),
            [pltpu.SemaphoreType.DMA] * _SNB,
            [pltpu.SemaphoreType.DMA] * _SNB,
            [pltpu.SemaphoreType.DMA] * _SNB,
            [pltpu.SemaphoreType.DMA] * _SNB,
        ],
    )
    def scatter_k(i_hbm, z_hbm, vecf_hbm, zero_hbm, out4, idx_v, row_v, vec_v,
                  table, sem_ld, sem_sc, sem_ix, sem_vc):
        core = lax.axis_index("c")
        sub = lax.axis_index("s")

        def issue_loads(p, zsel, k, b):
            e0 = sub * EPT + k * _SCC
            pltpu.async_copy(z_hbm.at[zsel, pl.ds(e0, _SCC), :], row_v[b],
                             sem_ld[b])
            pltpu.async_copy(i_hbm.at[pl.ds(e0, _SCC)], idx_v[b], sem_ix[b])
            pltpu.async_copy(vecf_hbm.at[pl.ds(3 * e0, 3 * _SCC)], vec_v[b],
                             sem_vc[b])

        def wait_loads(p, zsel, k, b):
            e0 = sub * EPT + k * _SCC
            pltpu.make_async_copy(z_hbm.at[zsel, pl.ds(e0, _SCC), :],
                                  row_v[b], sem_ld[b]).wait()
            pltpu.make_async_copy(i_hbm.at[pl.ds(e0, _SCC)], idx_v[b],
                                  sem_ix[b]).wait()
            pltpu.make_async_copy(vecf_hbm.at[pl.ds(3 * e0, 3 * _SCC)],
                                  vec_v[b], sem_vc[b]).wait()

        def wait_scatter(b):
            pltpu.make_async_copy(row_v[b], table.at[idx_v[b]],
                                  sem_sc[b]).wait()

        for q in range(2):
            p = 2 * core + q
            zsel = jnp.minimum(p, 1)
            d = jnp.maximum(p - 1, 0)

            @pl.when(sub == 0)
            def _zero():
                pltpu.sync_copy(zero_hbm, table)

            plsc.subcore_barrier()

            issue_loads(p, zsel, 0, 0)
            issue_loads(p, zsel, 1, 1)

            def group(g, carry):
                for b in range(_SNB):
                    k = _SNB * g + b
                    wait_loads(p, zsel, k, b)

                    @pl.when(p > 0)
                    def _scale():
                        def edge_body(e, carry2):
                            c = plsc.load_gather(
                                vec_v[b],
                                [jnp.full((16,), 3 * e, jnp.int32) + d])
                            for kk in range(H // 16):
                                row_v[b][e, pl.ds(kk * 16, 16)] = (
                                    row_v[b][e, pl.ds(kk * 16, 16)] * c)
                            return carry2

                        lax.fori_loop(0, _SCC, edge_body, 0)

                    pltpu.async_copy(row_v[b], table.at[idx_v[b]], sem_sc[b],
                                     add=True)

                    @pl.when(k >= 2)
                    def _():
                        wait_scatter((b + 2) % _SNB)

                    @pl.when(k + 2 < _SNCH)
                    def _():
                        issue_loads(p, zsel, k + 2, (b + 2) % _SNB)
                return carry

            lax.fori_loop(0, _SNCH // _SNB, group, 0)
            wait_scatter((_SNCH - 2) % _SNB)
            wait_scatter((_SNCH - 1) % _SNB)
            plsc.subcore_barrier()

            @pl.when(sub == 0)
            def _flush():
                pltpu.sync_copy(table, out4.at[p])

            plsc.subcore_barrier()

    return scatter_k


def kernel(s, v, edge_index, edge_rbf, edge_cutoff, edge_vec, W1, b1, W2, b2,
           Wr, br):
    i = edge_index[0].astype(jnp.int32)
    j = edge_index[1].astype(jnp.int32)
    n = s.shape[0]
    v2d = v.reshape(n, 3 * H)
    vecf = edge_vec.reshape(-1)

    sj, inner = _gather_stage()(j, s, v2d, vecf)
    z = _tc_stage(sj, inner, edge_rbf, edge_cutoff[:, None],
                  W1, b1[None, :], W2, b2[None, :], Wr, br[None, :])
    zero = jnp.zeros((n, H), jnp.float32)
    out4 = _scatter_stage()(i, z, vecf, zero)
    ds = out4[0]
    dv = jnp.transpose(out4[1:4], (1, 0, 2))
    return ds, dv


# speedup vs baseline: 1.5415x; 1.1257x over previous
import functools

import jax
import jax.numpy as jnp
from jax import lax
from jax.experimental import pallas as pl
from jax.experimental.pallas import tpu as pltpu
from jax.experimental.pallas import tpu_sc as plsc

N_NODES = 10000
N_EDGES = 320000
H = 128
NUM_RBF = 20

NC, NS = 2, 16
NW = NC * NS
EPW = N_EDGES // NW
EPT = N_EDGES // NS
GC = 40
SC_CHUNK = 80

def _mesh():
    return plsc.VectorSubcoreMesh(
        core_axis_name="c", subcore_axis_name="s", num_cores=NC, num_subcores=NS)


_GNB = 4


@functools.cache
def _gather_stage(ne):
    epw = ne // NW
    _GNCH = epw // GC

    @functools.partial(
        pl.kernel,
        out_type=[
            jax.ShapeDtypeStruct((ne, H), jnp.float32),
            jax.ShapeDtypeStruct((ne, H), jnp.float32),
        ],
        mesh=_mesh(),
        compiler_params=pltpu.CompilerParams(needs_layout_passes=False),
        scratch_types=[
            [pltpu.VMEM((GC,), jnp.int32)] * _GNB,
            [pltpu.VMEM((GC, H), jnp.float32)] * _GNB,
            [pltpu.VMEM((GC, 3 * H), jnp.float32)] * _GNB,
            [pltpu.VMEM((3 * GC,), jnp.float32)] * _GNB,
            [pltpu.VMEM((GC, H), jnp.float32)] * _GNB,
            [pltpu.SemaphoreType.DMA] * _GNB,
            [pltpu.SemaphoreType.DMA] * _GNB,
            [pltpu.SemaphoreType.DMA] * _GNB,
            [pltpu.SemaphoreType.DMA] * _GNB,
            [pltpu.SemaphoreType.DMA] * _GNB,
        ],
    )
    def gather_k(j_hbm, s_hbm, v_hbm, vecf_hbm, sj_out, inner_out, idx_v,
                 srow_v, vrow_v, vecc_v, ibuf_v, sem_s, sem_v, sem_c, sem_ws,
                 sem_wi):
        wid = lax.axis_index("s") * NC + lax.axis_index("c")
        base = wid * epw

        def issue(k, b):
            e0 = base + k * GC
            pltpu.sync_copy(j_hbm.at[pl.ds(e0, GC)], idx_v[b])
            pltpu.async_copy(s_hbm.at[idx_v[b]], srow_v[b], sem_s[b])
            pltpu.async_copy(v_hbm.at[idx_v[b]], vrow_v[b], sem_v[b])
            pltpu.async_copy(vecf_hbm.at[pl.ds(3 * e0, 3 * GC)], vecc_v[b],
                             sem_c[b])

        def wait_wb(k, b):
            e0 = base + k * GC
            pltpu.make_async_copy(srow_v[b], sj_out.at[pl.ds(e0, GC), :],
                                  sem_ws[b]).wait()
            pltpu.make_async_copy(ibuf_v[b], inner_out.at[pl.ds(e0, GC), :],
                                  sem_wi[b]).wait()

        def consume(k, b):
            e0 = base + k * GC
            pltpu.make_async_copy(s_hbm.at[idx_v[b]], srow_v[b],
                                  sem_s[b]).wait()
            pltpu.async_copy(srow_v[b], sj_out.at[pl.ds(e0, GC), :],
                             sem_ws[b])
            pltpu.make_async_copy(v_hbm.at[idx_v[b]], vrow_v[b],
                                  sem_v[b]).wait()
            pltpu.make_async_copy(vecf_hbm.at[pl.ds(3 * e0, 3 * GC)],
                                  vecc_v[b], sem_c[b]).wait()

            def edge_body(e, carry):
                cs = [
                    plsc.load_gather(
                        vecc_v[b],
                        [jnp.full((16,), 3 * e + d, jnp.int32)])
                    for d in range(3)
                ]
                for kk in range(H // 16):
                    acc = (vrow_v[b][e, pl.ds(kk * 16, 16)] * cs[0]
                           + vrow_v[b][e, pl.ds(H + kk * 16, 16)] * cs[1]
                           + vrow_v[b][e, pl.ds(2 * H + kk * 16, 16)] * cs[2])
                    ibuf_v[b][e, pl.ds(kk * 16, 16)] = acc
                return carry

            lax.fori_loop(0, GC, edge_body, 0)
            pltpu.async_copy(ibuf_v[b], inner_out.at[pl.ds(e0, GC), :],
                             sem_wi[b])

        issue(0, 0)
        issue(1, 1)

        def group(g, carry):
            for b in range(_GNB):
                k = _GNB * g + b

                @pl.when(k < _GNCH)
                def _():
                    consume(k, b)

                    @pl.when(k >= 2)
                    def _():
                        wait_wb(k - 2, (b + 2) % _GNB)

                    @pl.when(k + 2 < _GNCH)
                    def _():
                        issue(k + 2, (b + 2) % _GNB)

            return carry

        lax.fori_loop(0, (_GNCH + _GNB - 1) // _GNB, group, 0)
        wait_wb(_GNCH - 2, (_GNCH - 2) % _GNB)
        wait_wb(_GNCH - 1, (_GNCH - 1) % _GNB)

    return gather_k


_TCB = 3200


def _tc_body(sj_ref, in_ref, rbf_ref, cut_ref, w1_ref, b1_ref,
             w2_ref, b2_ref, wr_ref, br_ref, z_ref):
    sj = sj_ref[...]
    h = jnp.dot(sj, w1_ref[...], preferred_element_type=jnp.float32) + b1_ref[...]
    h = h * (1.0 / (1.0 + jnp.exp(-h)))
    h = jnp.dot(h, w2_ref[...], preferred_element_type=jnp.float32) + b2_ref[...]
    wt = jnp.dot(rbf_ref[...], wr_ref[...], preferred_element_type=jnp.float32)
    wt = (wt + br_ref[...]) * cut_ref[...]
    x = h * wt
    x_ss = x[:, :H]
    x_sv = x[:, H:2 * H]
    x_vv = x[:, 2 * H:]
    u = x_sv + in_ref[...] * x_vv
    z_ref[0] = x_ss
    z_ref[1] = u


def _tc_stage(sj, inner, rbf, cut, w1, b1, w2, b2, wr, br):
    ne = sj.shape[0]
    grid = (ne // _TCB,)
    return pl.pallas_call(
        _tc_body,
        grid=grid,
        in_specs=[
            pl.BlockSpec((_TCB, H), lambda e: (e, 0)),
            pl.BlockSpec((_TCB, H), lambda e: (e, 0)),
            pl.BlockSpec((_TCB, NUM_RBF), lambda e: (e, 0)),
            pl.BlockSpec((_TCB, 1), lambda e: (e, 0)),
            pl.BlockSpec((H, H), lambda e: (0, 0)),
            pl.BlockSpec((1, H), lambda e: (0, 0)),
            pl.BlockSpec((H, 3 * H), lambda e: (0, 0)),
            pl.BlockSpec((1, 3 * H), lambda e: (0, 0)),
            pl.BlockSpec((NUM_RBF, 3 * H), lambda e: (0, 0)),
            pl.BlockSpec((1, 3 * H), lambda e: (0, 0)),
        ],
        out_specs=pl.BlockSpec((2, _TCB, H), lambda e: (0, e, 0)),
        out_shape=jax.ShapeDtypeStruct((2, ne, H), jnp.float32),
    )(sj, inner, rbf, cut, w1, b1, w2, b2, wr, br)


_SCC = 40
_SNB = 4


@functools.cache
def _scatter_stage(ne):
    ept = ne // NS
    _SNCH = ept // _SCC

    @functools.partial(
        pl.kernel,
        out_type=jax.ShapeDtypeStruct((4, N_NODES, H), jnp.float32),
        mesh=_mesh(),
        compiler_params=pltpu.CompilerParams(needs_layout_passes=False),
        scratch_types=[
            [pltpu.VMEM((_SCC,), jnp.int32)] * _SNB,
            [pltpu.VMEM((_SCC, H), jnp.float32)] * _SNB,
            [pltpu.VMEM((3 * _SCC,), jnp.float32)] * _SNB,
            pltpu.VMEM_SHARED((N_NODES, H), jnp.float32),
            [pltpu.SemaphoreType.DMA] * _SNB,
            [pltpu.SemaphoreType.DMA] * _SNB,
            [pltpu.SemaphoreType.DMA] * _SNB,
            [pltpu.SemaphoreType.DMA] * _SNB,
        ],
    )
    def scatter_k(i_hbm, z_hbm, vecf_hbm, zero_hbm, out4, idx_v, row_v, vec_v,
                  table, sem_ld, sem_sc, sem_ix, sem_vc):
        core = lax.axis_index("c")
        sub = lax.axis_index("s")

        def issue_loads(p, zsel, k, b):
            e0 = sub * ept + k * _SCC
            pltpu.async_copy(z_hbm.at[zsel, pl.ds(e0, _SCC), :], row_v[b],
                             sem_ld[b])
            pltpu.async_copy(i_hbm.at[pl.ds(e0, _SCC)], idx_v[b], sem_ix[b])
            pltpu.async_copy(vecf_hbm.at[pl.ds(3 * e0, 3 * _SCC)], vec_v[b],
                             sem_vc[b])

        def wait_loads(p, zsel, k, b):
            e0 = sub * ept + k * _SCC
            pltpu.make_async_copy(z_hbm.at[zsel, pl.ds(e0, _SCC), :],
                                  row_v[b], sem_ld[b]).wait()
            pltpu.make_async_copy(i_hbm.at[pl.ds(e0, _SCC)], idx_v[b],
                                  sem_ix[b]).wait()
            pltpu.make_async_copy(vecf_hbm.at[pl.ds(3 * e0, 3 * _SCC)],
                                  vec_v[b], sem_vc[b]).wait()

        def wait_scatter(b):
            pltpu.make_async_copy(row_v[b], table.at[idx_v[b]],
                                  sem_sc[b]).wait()

        for q in range(2):
            p = 2 * core + q
            zsel = jnp.minimum(p, 1)
            d = jnp.maximum(p - 1, 0)

            @pl.when(sub == 0)
            def _zero():
                pltpu.sync_copy(zero_hbm, table)

            plsc.subcore_barrier()

            issue_loads(p, zsel, 0, 0)
            issue_loads(p, zsel, 1, 1)

            def group(g, carry):
                for b in range(_SNB):
                    k = _SNB * g + b

                    @pl.when(k < _SNCH)
                    def _chunk():
                        wait_loads(p, zsel, k, b)

                        @pl.when(p > 0)
                        def _scale():
                            def edge_body(e, carry2):
                                c = plsc.load_gather(
                                    vec_v[b],
                                    [jnp.full((16,), 3 * e, jnp.int32) + d])
                                for kk in range(H // 16):
                                    row_v[b][e, pl.ds(kk * 16, 16)] = (
                                        row_v[b][e, pl.ds(kk * 16, 16)] * c)
                                return carry2

                            lax.fori_loop(0, _SCC, edge_body, 0)

                        pltpu.async_copy(row_v[b], table.at[idx_v[b]],
                                         sem_sc[b], add=True)

                        @pl.when(k >= 2)
                        def _():
                            wait_scatter((b + 2) % _SNB)

                        @pl.when(k + 2 < _SNCH)
                        def _():
                            issue_loads(p, zsel, k + 2, (b + 2) % _SNB)
                return carry

            lax.fori_loop(0, (_SNCH + _SNB - 1) // _SNB, group, 0)
            wait_scatter((_SNCH - 2) % _SNB)
            wait_scatter((_SNCH - 1) % _SNB)
            plsc.subcore_barrier()

            @pl.when(sub == 0)
            def _flush():
                pltpu.sync_copy(table, out4.at[p])

            plsc.subcore_barrier()

    return scatter_k


def kernel(s, v, edge_index, edge_rbf, edge_cutoff, edge_vec, W1, b1, W2, b2,
           Wr, br):
    i = edge_index[0].astype(jnp.int32)
    j = edge_index[1].astype(jnp.int32)
    n = s.shape[0]
    v2d = v.reshape(n, 3 * H)
    vecf = edge_vec.reshape(-1)
    zero = jnp.zeros((n, H), jnp.float32)

    nh = N_EDGES // 2
    out4 = None
    for hlo in (0, nh):
        jh = lax.dynamic_slice_in_dim(j, hlo, nh)
        ih = lax.dynamic_slice_in_dim(i, hlo, nh)
        vech = lax.dynamic_slice_in_dim(vecf, 3 * hlo, 3 * nh)
        rbfh = lax.dynamic_slice_in_dim(edge_rbf, hlo, nh)
        cuth = lax.dynamic_slice_in_dim(edge_cutoff, hlo, nh)
        sj, inner = _gather_stage(nh)(jh, s, v2d, vech)
        z = _tc_stage(sj, inner, rbfh, cuth[:, None],
                      W1, b1[None, :], W2, b2[None, :], Wr, br[None, :])
        part = _scatter_stage(nh)(ih, z, vech, zero)
        out4 = part if out4 is None else out4 + part
    ds = out4[0]
    dv = jnp.transpose(out4[1:4], (1, 0, 2))
    return ds, dv


# speedup vs baseline: 1.6869x; 1.0943x over previous
import functools

import jax
import jax.numpy as jnp
from jax import lax
from jax.experimental import pallas as pl
from jax.experimental.pallas import tpu as pltpu
from jax.experimental.pallas import tpu_sc as plsc

N_NODES = 10000
N_EDGES = 320000
H = 128
NUM_RBF = 20

NC, NS = 2, 16
NW = NC * NS
EPW = N_EDGES // NW
EPT = N_EDGES // NS
GC = 40
SC_CHUNK = 80

def _mesh():
    return plsc.VectorSubcoreMesh(
        core_axis_name="c", subcore_axis_name="s", num_cores=NC, num_subcores=NS)


_GNB = 4


@functools.cache
def _gather_stage(ne):
    epw = ne // NW
    _GNCH = epw // GC

    @functools.partial(
        pl.kernel,
        out_type=[
            jax.ShapeDtypeStruct((ne, H), jnp.float32),
            jax.ShapeDtypeStruct((ne, H), jnp.float32),
        ],
        mesh=_mesh(),
        compiler_params=pltpu.CompilerParams(needs_layout_passes=False),
        scratch_types=[
            [pltpu.VMEM((GC,), jnp.int32)] * _GNB,
            [pltpu.VMEM((GC, H), jnp.float32)] * _GNB,
            [pltpu.VMEM((GC, 3 * H), jnp.float32)] * _GNB,
            [pltpu.VMEM((3 * GC,), jnp.float32)] * _GNB,
            [pltpu.VMEM((GC, H), jnp.float32)] * _GNB,
            [pltpu.SemaphoreType.DMA] * _GNB,
            [pltpu.SemaphoreType.DMA] * _GNB,
            [pltpu.SemaphoreType.DMA] * _GNB,
            [pltpu.SemaphoreType.DMA] * _GNB,
            [pltpu.SemaphoreType.DMA] * _GNB,
        ],
    )
    def gather_k(j_hbm, s_hbm, v_hbm, vecf_hbm, sj_out, inner_out, idx_v,
                 srow_v, vrow_v, vecc_v, ibuf_v, sem_s, sem_v, sem_c, sem_ws,
                 sem_wi):
        wid = lax.axis_index("s") * NC + lax.axis_index("c")
        base = wid * epw

        def issue(k, b):
            e0 = base + k * GC
            pltpu.sync_copy(j_hbm.at[pl.ds(e0, GC)], idx_v[b])
            pltpu.async_copy(s_hbm.at[idx_v[b]], srow_v[b], sem_s[b])
            pltpu.async_copy(v_hbm.at[idx_v[b]], vrow_v[b], sem_v[b])
            pltpu.async_copy(vecf_hbm.at[pl.ds(3 * e0, 3 * GC)], vecc_v[b],
                             sem_c[b])

        def wait_wb(k, b):
            e0 = base + k * GC
            pltpu.make_async_copy(srow_v[b], sj_out.at[pl.ds(e0, GC), :],
                                  sem_ws[b]).wait()
            pltpu.make_async_copy(ibuf_v[b], inner_out.at[pl.ds(e0, GC), :],
                                  sem_wi[b]).wait()

        def consume(k, b):
            e0 = base + k * GC
            pltpu.make_async_copy(s_hbm.at[idx_v[b]], srow_v[b],
                                  sem_s[b]).wait()
            pltpu.async_copy(srow_v[b], sj_out.at[pl.ds(e0, GC), :],
                             sem_ws[b])
            pltpu.make_async_copy(v_hbm.at[idx_v[b]], vrow_v[b],
                                  sem_v[b]).wait()
            pltpu.make_async_copy(vecf_hbm.at[pl.ds(3 * e0, 3 * GC)],
                                  vecc_v[b], sem_c[b]).wait()

            def edge_body(e, carry):
                cs = [
                    plsc.load_gather(
                        vecc_v[b],
                        [jnp.full((16,), 3 * e + d, jnp.int32)])
                    for d in range(3)
                ]
                for kk in range(H // 16):
                    acc = (vrow_v[b][e, pl.ds(kk * 16, 16)] * cs[0]
                           + vrow_v[b][e, pl.ds(H + kk * 16, 16)] * cs[1]
                           + vrow_v[b][e, pl.ds(2 * H + kk * 16, 16)] * cs[2])
                    ibuf_v[b][e, pl.ds(kk * 16, 16)] = acc
                return carry

            lax.fori_loop(0, GC, edge_body, 0)
            pltpu.async_copy(ibuf_v[b], inner_out.at[pl.ds(e0, GC), :],
                             sem_wi[b])

        issue(0, 0)
        issue(1, 1)

        def group(g, carry):
            for b in range(_GNB):
                k = _GNB * g + b

                @pl.when(k < _GNCH)
                def _():
                    consume(k, b)

                    @pl.when(k >= 2)
                    def _():
                        wait_wb(k - 2, (b + 2) % _GNB)

                    @pl.when(k + 2 < _GNCH)
                    def _():
                        issue(k + 2, (b + 2) % _GNB)

            return carry

        lax.fori_loop(0, (_GNCH + _GNB - 1) // _GNB, group, 0)
        wait_wb(_GNCH - 2, (_GNCH - 2) % _GNB)
        wait_wb(_GNCH - 1, (_GNCH - 1) % _GNB)

    return gather_k


_TCB = 3200


def _tc_body(sj_ref, in_ref, rbf_ref, cut_ref, w1_ref, b1_ref,
             w2_ref, b2_ref, wr_ref, br_ref, z_ref):
    sj = sj_ref[...]
    h = jnp.dot(sj, w1_ref[...], preferred_element_type=jnp.float32) + b1_ref[...]
    h = h * (1.0 / (1.0 + jnp.exp(-h)))
    h = jnp.dot(h, w2_ref[...], preferred_element_type=jnp.float32) + b2_ref[...]
    wt = jnp.dot(rbf_ref[...], wr_ref[...], preferred_element_type=jnp.float32)
    wt = (wt + br_ref[...]) * cut_ref[...]
    x = h * wt
    x_ss = x[:, :H]
    x_sv = x[:, H:2 * H]
    x_vv = x[:, 2 * H:]
    u = x_sv + in_ref[...] * x_vv
    z_ref[0] = x_ss
    z_ref[1] = u


def _tc_stage(sj, inner, rbf, cut, w1, b1, w2, b2, wr, br):
    ne = sj.shape[0]
    grid = (ne // _TCB,)
    return pl.pallas_call(
        _tc_body,
        grid=grid,
        in_specs=[
            pl.BlockSpec((_TCB, H), lambda e: (e, 0)),
            pl.BlockSpec((_TCB, H), lambda e: (e, 0)),
            pl.BlockSpec((_TCB, NUM_RBF), lambda e: (e, 0)),
            pl.BlockSpec((_TCB, 1), lambda e: (e, 0)),
            pl.BlockSpec((H, H), lambda e: (0, 0)),
            pl.BlockSpec((1, H), lambda e: (0, 0)),
            pl.BlockSpec((H, 3 * H), lambda e: (0, 0)),
            pl.BlockSpec((1, 3 * H), lambda e: (0, 0)),
            pl.BlockSpec((NUM_RBF, 3 * H), lambda e: (0, 0)),
            pl.BlockSpec((1, 3 * H), lambda e: (0, 0)),
        ],
        out_specs=pl.BlockSpec((2, _TCB, H), lambda e: (0, e, 0)),
        out_shape=jax.ShapeDtypeStruct((2, ne, H), jnp.float32),
    )(sj, inner, rbf, cut, w1, b1, w2, b2, wr, br)


_SCC = 80
_SNB = 4


@functools.cache
def _scatter_stage(ne):
    ept = ne // NS
    _SNCH = ept // _SCC

    @functools.partial(
        pl.kernel,
        out_type=jax.ShapeDtypeStruct((4, N_NODES, H), jnp.float32),
        mesh=_mesh(),
        compiler_params=pltpu.CompilerParams(needs_layout_passes=False),
        scratch_types=[
            [pltpu.VMEM((_SCC,), jnp.int32)] * _SNB,
            [pltpu.VMEM((_SCC, H), jnp.float32)] * _SNB,
            [pltpu.VMEM((3 * _SCC,), jnp.float32)] * _SNB,
            pltpu.VMEM_SHARED((N_NODES, H), jnp.float32),
            [pltpu.SemaphoreType.DMA] * _SNB,
            [pltpu.SemaphoreType.DMA] * _SNB,
            [pltpu.SemaphoreType.DMA] * _SNB,
            [pltpu.SemaphoreType.DMA] * _SNB,
        ],
    )
    def scatter_k(i_hbm, z_hbm, vecf_hbm, zero_hbm, out4, idx_v, row_v, vec_v,
                  table, sem_ld, sem_sc, sem_ix, sem_vc):
        core = lax.axis_index("c")
        sub = lax.axis_index("s")

        def issue_loads(p, zsel, k, b):
            e0 = sub * ept + k * _SCC
            pltpu.async_copy(z_hbm.at[zsel, pl.ds(e0, _SCC), :], row_v[b],
                             sem_ld[b])
            pltpu.async_copy(i_hbm.at[pl.ds(e0, _SCC)], idx_v[b], sem_ix[b])
            pltpu.async_copy(vecf_hbm.at[pl.ds(3 * e0, 3 * _SCC)], vec_v[b],
                             sem_vc[b])

        def wait_loads(p, zsel, k, b):
            e0 = sub * ept + k * _SCC
            pltpu.make_async_copy(z_hbm.at[zsel, pl.ds(e0, _SCC), :],
                                  row_v[b], sem_ld[b]).wait()
            pltpu.make_async_copy(i_hbm.at[pl.ds(e0, _SCC)], idx_v[b],
                                  sem_ix[b]).wait()
            pltpu.make_async_copy(vecf_hbm.at[pl.ds(3 * e0, 3 * _SCC)],
                                  vec_v[b], sem_vc[b]).wait()

        def wait_scatter(b):
            pltpu.make_async_copy(row_v[b], table.at[idx_v[b]],
                                  sem_sc[b]).wait()

        for q in range(2):
            p = 2 * core + q
            zsel = jnp.minimum(p, 1)
            d = jnp.maximum(p - 1, 0)

            @pl.when(sub == 0)
            def _zero():
                pltpu.sync_copy(zero_hbm, table)

            plsc.subcore_barrier()

            issue_loads(p, zsel, 0, 0)
            issue_loads(p, zsel, 1, 1)

            def group(g, carry):
                for b in range(_SNB):
                    k = _SNB * g + b

                    @pl.when(k < _SNCH)
                    def _chunk():
                        wait_loads(p, zsel, k, b)

                        @pl.when(p > 0)
                        def _scale():
                            def edge_body(e, carry2):
                                c = plsc.load_gather(
                                    vec_v[b],
                                    [jnp.full((16,), 3 * e, jnp.int32) + d])
                                for kk in range(H // 16):
                                    row_v[b][e, pl.ds(kk * 16, 16)] = (
                                        row_v[b][e, pl.ds(kk * 16, 16)] * c)
                                return carry2

                            lax.fori_loop(0, _SCC, edge_body, 0)

                        pltpu.async_copy(row_v[b], table.at[idx_v[b]],
                                         sem_sc[b], add=True)

                        @pl.when(k >= 2)
                        def _():
                            wait_scatter((b + 2) % _SNB)

                        @pl.when(k + 2 < _SNCH)
                        def _():
                            issue_loads(p, zsel, k + 2, (b + 2) % _SNB)
                return carry

            lax.fori_loop(0, (_SNCH + _SNB - 1) // _SNB, group, 0)
            wait_scatter((_SNCH - 2) % _SNB)
            wait_scatter((_SNCH - 1) % _SNB)
            plsc.subcore_barrier()

            @pl.when(sub == 0)
            def _flush():
                pltpu.sync_copy(table, out4.at[p])

            plsc.subcore_barrier()

    return scatter_k


def kernel(s, v, edge_index, edge_rbf, edge_cutoff, edge_vec, W1, b1, W2, b2,
           Wr, br):
    i = edge_index[0].astype(jnp.int32)
    j = edge_index[1].astype(jnp.int32)
    n = s.shape[0]
    v2d = v.reshape(n, 3 * H)
    vecf = edge_vec.reshape(-1)
    zero = jnp.zeros((n, H), jnp.float32)

    nh = N_EDGES // 2
    out4 = None
    for hlo in (0, nh):
        jh = lax.dynamic_slice_in_dim(j, hlo, nh)
        ih = lax.dynamic_slice_in_dim(i, hlo, nh)
        vech = lax.dynamic_slice_in_dim(vecf, 3 * hlo, 3 * nh)
        rbfh = lax.dynamic_slice_in_dim(edge_rbf, hlo, nh)
        cuth = lax.dynamic_slice_in_dim(edge_cutoff, hlo, nh)
        sj, inner = _gather_stage(nh)(jh, s, v2d, vech)
        z = _tc_stage(sj, inner, rbfh, cuth[:, None],
                      W1, b1[None, :], W2, b2[None, :], Wr, br[None, :])
        part = _scatter_stage(nh)(ih, z, vech, zero)
        out4 = part if out4 is None else out4 + part
    ds = out4[0]
    dv = jnp.transpose(out4[1:4], (1, 0, 2))
    return ds, dv
